# baseline (speedup 1.0000x reference)
import functools

import jax
import jax.numpy as jnp
from jax.experimental import pallas as pl
from jax.experimental.pallas import tpu as pltpu

HEADS = 4
HID = 32
F = HEADS * HID


def _node_proj_body(h0_ref, w_ref, as_ref, ad_ref, h_ref, asrc_ref, adst_ref):
    h = jnp.dot(h0_ref[...], w_ref[...], preferred_element_type=jnp.float32)
    h_ref[...] = h
    asrc_ref[...] = jnp.dot(h, as_ref[...], preferred_element_type=jnp.float32)
    adst_ref[...] = jnp.dot(h, ad_ref[...], preferred_element_type=jnp.float32)


def _node_proj(h0, W, As, Ad, bn=1000):
    n, k = h0.shape
    grid = n // bn
    return pl.pallas_call(
        _node_proj_body,
        grid=(grid,),
        in_specs=[
            pl.BlockSpec((bn, k), lambda i: (i, 0)),
            pl.BlockSpec((k, F), lambda i: (0, 0)),
            pl.BlockSpec((F, HEADS), lambda i: (0, 0)),
            pl.BlockSpec((F, HEADS), lambda i: (0, 0)),
        ],
        out_specs=[
            pl.BlockSpec((bn, F), lambda i: (i, 0)),
            pl.BlockSpec((bn, HEADS), lambda i: (i, 0)),
            pl.BlockSpec((bn, HEADS), lambda i: (i, 0)),
        ],
        out_shape=[
            jax.ShapeDtypeStruct((n, F), jnp.float32),
            jax.ShapeDtypeStruct((n, HEADS), jnp.float32),
            jax.ShapeDtypeStruct((n, HEADS), jnp.float32),
        ],
    )(h0, W, As, Ad)


def _decoder_body(hs_ref, hd_ref, ea_ref, w1s_ref, w1d_ref, w1e_ref, b1_ref,
                  w2_ref, b2_ref, w3_ref, b3_ref, out_ref):
    z = jnp.dot(hs_ref[...], w1s_ref[...], preferred_element_type=jnp.float32)
    z += jnp.dot(hd_ref[...], w1d_ref[...], preferred_element_type=jnp.float32)
    z += jnp.dot(ea_ref[...], w1e_ref[...], preferred_element_type=jnp.float32)
    z = jnp.maximum(z + b1_ref[...], 0.0)
    z = jnp.maximum(
        jnp.dot(z, w2_ref[...], preferred_element_type=jnp.float32) + b2_ref[...], 0.0)
    out_ref[...] = (
        jnp.dot(z, w3_ref[...], preferred_element_type=jnp.float32) + b3_ref[...])


def _decoder(hs, hd, ea, Wd1, bd1, Wd2, bd2, Wd3, bd3, be=8000):
    e = hs.shape[0]
    grid = e // be
    w1s, w1d, w1e = Wd1[:HID], Wd1[HID:2 * HID], Wd1[2 * HID:]
    return pl.pallas_call(
        _decoder_body,
        grid=(grid,),
        in_specs=[
            pl.BlockSpec((be, HID), lambda i: (i, 0)),
            pl.BlockSpec((be, HID), lambda i: (i, 0)),
            pl.BlockSpec((be, 5), lambda i: (i, 0)),
            pl.BlockSpec((HID, 2 * HID), lambda i: (0, 0)),
            pl.BlockSpec((HID, 2 * HID), lambda i: (0, 0)),
            pl.BlockSpec((5, 2 * HID), lambda i: (0, 0)),
            pl.BlockSpec((1, 2 * HID), lambda i: (0, 0)),
            pl.BlockSpec((2 * HID, HID), lambda i: (0, 0)),
            pl.BlockSpec((1, HID), lambda i: (0, 0)),
            pl.BlockSpec((HID, 4), lambda i: (0, 0)),
            pl.BlockSpec((1, 4), lambda i: (0, 0)),
        ],
        out_specs=pl.BlockSpec((be, 4), lambda i: (i, 0)),
        out_shape=jax.ShapeDtypeStruct((e, 4), jnp.float32),
    )(hs, hd, ea, w1s, w1d, w1e, bd1[None], Wd2, bd2[None], Wd3, bd3[None])


def _block_diag_att(att):
    m = jnp.zeros((HEADS, HID, HEADS), dtype=att.dtype)
    m = m.at[jnp.arange(HEADS), :, jnp.arange(HEADS)].set(att[0])
    return m.reshape(F, HEADS)


def _gat_layer(h0, src, dst, W, att_src, att_dst, bias, concat):
    n = h0.shape[0]
    As = _block_diag_att(att_src)
    Ad = _block_diag_att(att_dst)
    H, a_src, a_dst = _node_proj(h0, W, As, Ad)

    alpha = a_src[src] + a_dst[dst]
    alpha = jax.nn.leaky_relu(alpha, 0.2)
    amax = jax.ops.segment_max(alpha, dst, num_segments=n)
    ex = jnp.exp(alpha - amax[dst])
    denom = jax.ops.segment_sum(ex, dst, num_segments=n)
    coef = ex / (denom[dst] + 1e-16)
    h3 = H.reshape(n, HEADS, HID)
    msg = h3[src] * coef[:, :, None]
    out = jax.ops.segment_sum(msg, dst, num_segments=n)
    if concat:
        out = out.reshape(n, F)
    else:
        out = out.mean(axis=1)
    return out + bias


def kernel(x, edge_index, edge_attr, u, W1, att1_src, att1_dst, b1,
           W2, att2_src, att2_dst, b2, Wd1, bd1, Wd2, bd2, Wd3, bd3):
    n = x.shape[0]
    loop = jnp.arange(n, dtype=edge_index.dtype)
    src = jnp.concatenate([edge_index[0], loop])
    dst = jnp.concatenate([edge_index[1], loop])

    u_node = jnp.broadcast_to(u, (n, u.shape[1]))
    h0 = jnp.concatenate([x, u_node], axis=-1)
    h = jax.nn.elu(_gat_layer(h0, src, dst, W1, att1_src, att1_dst, b1, True))
    h = jax.nn.elu(_gat_layer(h, src, dst, W2, att2_src, att2_dst, b2, False))

    esrc, edst = edge_index[0], edge_index[1]
    hs = h[esrc]
    hd = h[edst]
    return _decoder(hs, hd, edge_attr, Wd1, bd1, Wd2, bd2, Wd3, bd3)


# speedup vs baseline: 23.6867x; 23.6867x over previous
import functools

import jax
import jax.numpy as jnp
from jax import lax
from jax.experimental import pallas as pl
from jax.experimental.pallas import tpu as pltpu
from jax.experimental.pallas import tpu_sc as plsc

HEADS = 4
HID = 32
F = HEADS * HID
L = 16
NC = 2
NS = 16
NW = NC * NS
GRP = 80


def _mesh():
    return plsc.VectorSubcoreMesh(
        core_axis_name="c", subcore_axis_name="s",
        num_cores=NC, num_subcores=NS)


def _proj_body(h0_ref, w_ref, as_ref, ad_ref,
               h_ref, asp_ref, adp_ref, ms_ref, md_ref):
    i = pl.program_id(0)
    h = jnp.dot(h0_ref[...], w_ref[...], preferred_element_type=jnp.float32)
    h_ref[...] = h
    a_s = jnp.dot(h, as_ref[...], preferred_element_type=jnp.float32)
    a_d = jnp.dot(h, ad_ref[...], preferred_element_type=jnp.float32)
    bn = a_s.shape[0]
    pad = jnp.zeros((bn, L - HEADS), jnp.float32)
    asp_ref[...] = jnp.concatenate([a_s, pad], axis=1)
    adp_ref[...] = jnp.concatenate([a_d, pad], axis=1)
    bs = jnp.max(a_s, axis=0, keepdims=True)
    bd = jnp.max(a_d, axis=0, keepdims=True)

    @pl.when(i == 0)
    def _():
        ms_ref[...] = bs
        md_ref[...] = bd

    @pl.when(i != 0)
    def _():
        ms_ref[...] = jnp.maximum(ms_ref[...], bs)
        md_ref[...] = jnp.maximum(md_ref[...], bd)


def _proj(h0, W, As, Ad, bn=1000):
    n, k = h0.shape
    return pl.pallas_call(
        _proj_body,
        grid=(n // bn,),
        in_specs=[
            pl.BlockSpec((bn, k), lambda i: (i, 0)),
            pl.BlockSpec((k, F), lambda i: (0, 0)),
            pl.BlockSpec((F, HEADS), lambda i: (0, 0)),
            pl.BlockSpec((F, HEADS), lambda i: (0, 0)),
        ],
        out_specs=[
            pl.BlockSpec((bn, F), lambda i: (i, 0)),
            pl.BlockSpec((bn, L), lambda i: (i, 0)),
            pl.BlockSpec((bn, L), lambda i: (i, 0)),
            pl.BlockSpec((1, HEADS), lambda i: (0, 0)),
            pl.BlockSpec((1, HEADS), lambda i: (0, 0)),
        ],
        out_shape=[
            jax.ShapeDtypeStruct((n, F), jnp.float32),
            jax.ShapeDtypeStruct((n, L), jnp.float32),
            jax.ShapeDtypeStruct((n, L), jnp.float32),
            jax.ShapeDtypeStruct((1, HEADS), jnp.float32),
            jax.ShapeDtypeStruct((1, HEADS), jnp.float32),
        ],
    )(h0, W, As, Ad)


def _stripe_zero(z_h, acc, sid, s0, tail):
    pltpu.sync_copy(z_h.at[pl.ds(sid * s0, s0)], acc.at[pl.ds(sid * s0, s0)])

    @pl.when(sid == NS - 1)
    def _():
        pltpu.sync_copy(z_h.at[pl.ds(NS * s0, tail)],
                        acc.at[pl.ds(NS * s0, tail)])


def _stripe_drain(acc, dst_ref, sid, s0, tail):
    pltpu.sync_copy(acc.at[pl.ds(sid * s0, s0)],
                    dst_ref.at[pl.ds(sid * s0, s0)])

    @pl.when(sid == NS - 1)
    def _():
        pltpu.sync_copy(acc.at[pl.ds(NS * s0, tail)],
                        dst_ref.at[pl.ds(NS * s0, tail)])


def _load_dst_groups(dst_h, dstb, base, ng, sem):
    def fire(g, _):
        pltpu.async_copy(dst_h.at[pl.ds(base + g * GRP, GRP)],
                         dstb.at[g], sem)
        return 0
    lax.fori_loop(0, ng, fire, 0)

    def drain(g, _):
        pltpu.make_async_copy(dst_h.at[pl.ds(base, GRP)],
                              dstb.at[0], sem).wait()
        return 0
    lax.fori_loop(0, ng, drain, 0)


def _range_idx(dstb, dstb2, lo, n2, ng):
    kk = GRP // L

    def body(q, _):
        g = q // kk
        k = q % kk
        dv = dstb[g, pl.ds(k * L, L)]
        ok = (dv >= lo) & (dv < lo + n2)
        dstb2[g, pl.ds(k * L, L)] = jnp.where(ok, dv - lo, n2)
        return 0
    lax.fori_loop(0, ng * kk, body, 0)


def _fire_gathers(table_h, idx_ref, rows_ref, ng, sem):
    def fire(g, _):
        pltpu.async_copy(table_h.at[idx_ref.at[pl.ds(g * GRP, GRP)]],
                         rows_ref.at[pl.ds(g * GRP, GRP)], sem)
        return 0
    lax.fori_loop(0, ng, fire, 0)


def _drain_gathers(table_h, idx_ref, rows_ref, ng, sem):
    def drain(g, _):
        pltpu.make_async_copy(table_h.at[idx_ref.at[pl.ds(0, GRP)]],
                              rows_ref.at[pl.ds(0, GRP)], sem).wait()
        return 0
    lax.fori_loop(0, ng, drain, 0)


def _scatter_add_groups(rows_ref, acc, idx2_ref, ng):
    def scat(g, _):
        pltpu.sync_copy(rows_ref.at[pl.ds(g * GRP, GRP)],
                        acc.at[idx2_ref.at[g]], add=True)
        return 0
    lax.fori_loop(0, ng, scat, 0)


def _sc_pass_a(src, dst, asp, adp, cvec, zeros16):
    e = src.shape[0]
    n = asp.shape[0]
    n2 = n // 2
    ew = e // NW
    c = 400
    ng = c // GRP
    nch = ew // c
    s0 = (n2 // NS) & ~7
    tail = n2 - NS * s0

    @functools.partial(
        pl.kernel,
        out_type=[
            jax.ShapeDtypeStruct((HEADS * e,), jnp.float32),
            jax.ShapeDtypeStruct((NC, n, L), jnp.float32),
        ],
        mesh=_mesh(),
        compiler_params=pltpu.CompilerParams(use_tc_tiling_on_sc=False),
        scratch_types=[
            pltpu.VMEM((c,), jnp.int32),
            pltpu.VMEM((ng, GRP), jnp.int32),
            pltpu.VMEM((ng, GRP), jnp.int32),
            pltpu.VMEM((c, L), jnp.float32),
            pltpu.VMEM((c, L), jnp.float32),
            pltpu.VMEM((c, L), jnp.float32),
            pltpu.VMEM((HEADS * c,), jnp.float32),
            pltpu.VMEM((L,), jnp.float32),
            pltpu.VMEM_SHARED((n2 + 8, L), jnp.float32),
            pltpu.SemaphoreType.DMA,
            pltpu.SemaphoreType.DMA,
        ],
    )
    def k(src_h, dst_h, asp_h, adp_h, cvec_h, z_h, exT_h, dp_h,
          srcb, dstb, dstb2, asb, adb, exb, exTb, cb, acc, s1, s2):
        cid = lax.axis_index("c")
        sid = lax.axis_index("s")
        wid = sid * NC + cid
        pltpu.sync_copy(cvec_h, cb)
        cv = cb[...]
        iota = lax.iota(jnp.int32, L)
        iotam4 = iota % HEADS
        m4 = iota < 4
        m8 = iota < 8
        m12 = iota < 12

        for r in range(2):
            lo = r * n2
            _stripe_zero(z_h, acc, sid, s0, tail + 8)
            plsc.subcore_barrier()

            def chunk(i, _):
                base = wid * ew + i * c
                pltpu.sync_copy(src_h.at[pl.ds(base, c)], srcb)
                _load_dst_groups(dst_h, dstb, base, ng, s2)
                _fire_gathers(asp_h, srcb, asb, ng, s1)

                def fire_ad(g, _):
                    pltpu.async_copy(adp_h.at[dstb.at[g]],
                                     adb.at[pl.ds(g * GRP, GRP)], s2)
                    return 0
                lax.fori_loop(0, ng, fire_ad, 0)
                _range_idx(dstb, dstb2, lo, n2, ng)
                _drain_gathers(asp_h, srcb, asb, ng, s1)

                def drain_ad(g, _):
                    pltpu.make_async_copy(adp_h.at[dstb.at[0]],
                                          adb.at[pl.ds(0, GRP)], s2).wait()
                    return 0
                lax.fori_loop(0, ng, drain_ad, 0)

                def body(jj, _):
                    gs = []
                    for t in range(4):
                        j = jj * 4 + t
                        s = asb[j] + adb[j]
                        v = jnp.exp(jnp.maximum(s, 0.2 * s) - cv)
                        exb[j] = v
                        if r == 0:
                            gs.append(
                                v.at[iotam4].get(mode="promise_in_bounds"))
                    if r == 0:
                        q = jnp.where(m4, gs[0],
                                      jnp.where(m8, gs[1],
                                                jnp.where(m12, gs[2], gs[3])))
                        exTb[pl.ds(jj * L, L)] = q
                    return 0
                lax.fori_loop(0, c // 4, body, 0)

                if r == 0:
                    pltpu.sync_copy(exTb,
                                    exT_h.at[pl.ds(base * HEADS, c * HEADS)])
                _scatter_add_groups(exb, acc, dstb2, ng)
                return 0
            lax.fori_loop(0, nch, chunk, 0)
            plsc.subcore_barrier()
            _stripe_drain(acc, dp_h.at[cid].at[pl.ds(lo, n2)], sid, s0, tail)
            plsc.subcore_barrier()

    return k(src, dst, asp, adp, cvec, zeros16)


def _sc_pass_b(src, dst, exT, h8, zeros16):
    e = src.shape[0]
    n = h8.shape[0] // 8
    n2 = n // 2
    ew = e // NW
    c = 2000
    ng = c // GRP
    nch = ew // c
    s0 = (n2 // NS) & ~7
    tail = n2 - NS * s0

    @functools.partial(
        pl.kernel,
        out_type=jax.ShapeDtypeStruct((8, NC, n, L), jnp.float32),
        mesh=_mesh(),
        compiler_params=pltpu.CompilerParams(use_tc_tiling_on_sc=False),
        scratch_types=[
            pltpu.VMEM((c,), jnp.int32),
            pltpu.VMEM((ng, GRP), jnp.int32),
            pltpu.VMEM((ng, GRP), jnp.int32),
            pltpu.VMEM((c,), jnp.int32),
            pltpu.VMEM((HEADS * c,), jnp.float32),
            pltpu.VMEM((c, L), jnp.float32),
            pltpu.VMEM_SHARED((n2 + 8, L), jnp.float32),
            pltpu.SemaphoreType.DMA,
            pltpu.SemaphoreType.DMA,
        ],
    )
    def k(src_h, dst_h, exT_h, h8_h, z_h, out_h,
          srcb, dstb, dstb2, idxb, exw, rows, acc, s1, s2):
        cid = lax.axis_index("c")
        sid = lax.axis_index("s")
        wid = sid * NC + cid
        for slot in range(8):
            hd = slot // 2
            for r in range(2):
                lo = r * n2
                _stripe_zero(z_h, acc, sid, s0, tail + 8)
                plsc.subcore_barrier()

                def chunk(i, _):
                    base = wid * ew + i * c
                    pltpu.sync_copy(src_h.at[pl.ds(base, c)], srcb)
                    _load_dst_groups(dst_h, dstb, base, ng, s2)
                    pltpu.sync_copy(
                        exT_h.at[pl.ds(base * HEADS, c * HEADS)], exw)

                    def mkidx(g, _):
                        v = srcb[pl.ds(g * L, L)]
                        idxb[pl.ds(g * L, L)] = v * 8 + slot
                        return 0
                    lax.fori_loop(0, c // L, mkidx, 0)
                    _fire_gathers(h8_h, idxb, rows, ng, s1)
                    _range_idx(dstb, dstb2, lo, n2, ng)
                    _drain_gathers(h8_h, idxb, rows, ng, s1)

                    def scale(jj, _):
                        ex16 = exw[pl.ds(jj * L, L)]
                        for t in range(4):
                            j = jj * 4 + t
                            rows[j] = rows[j] * ex16[t * HEADS + hd]
                        return 0
                    lax.fori_loop(0, c // 4, scale, 0)
                    _scatter_add_groups(rows, acc, dstb2, ng)
                    return 0
                lax.fori_loop(0, nch, chunk, 0)
                plsc.subcore_barrier()
                _stripe_drain(acc, out_h.at[slot, cid].at[pl.ds(lo, n2)],
                              sid, s0, tail)
                plsc.subcore_barrier()

    return k(src, dst, exT, h8, zeros16)


def _sc_pass_d(src, dst, hfin):
    e = src.shape[0]
    ew = e // NW
    c = 400
    ng = c // GRP
    nch = ew // c

    @functools.partial(
        pl.kernel,
        out_type=[
            jax.ShapeDtypeStruct((e, HID), jnp.float32),
            jax.ShapeDtypeStruct((e, HID), jnp.float32),
        ],
        mesh=_mesh(),
        compiler_params=pltpu.CompilerParams(
            use_tc_tiling_on_sc=False, internal_scratch_in_bytes=1 << 16),
        scratch_types=[
            pltpu.VMEM((c,), jnp.int32),
            pltpu.VMEM((c,), jnp.int32),
            pltpu.VMEM((c, HID), jnp.float32),
            pltpu.VMEM((c, HID), jnp.float32),
            pltpu.SemaphoreType.DMA,
            pltpu.SemaphoreType.DMA,
        ],
    )
    def k(src_h, dst_h, hf_h, hs_h, hd_h, srcb, dstb, rs, rd, s1, s2):
        cid = lax.axis_index("c")
        sid = lax.axis_index("s")
        wid = sid * NC + cid

        def chunk(i, _):
            base = wid * ew + i * c
            pltpu.sync_copy(src_h.at[pl.ds(base, c)], srcb)
            pltpu.sync_copy(dst_h.at[pl.ds(base, c)], dstb)
            cps = []
            for g in range(ng):
                cps.append(pltpu.async_copy(
                    hf_h.at[srcb.at[pl.ds(g * GRP, GRP)]],
                    rs.at[pl.ds(g * GRP, GRP)], s1))
                cps.append(pltpu.async_copy(
                    hf_h.at[dstb.at[pl.ds(g * GRP, GRP)]],
                    rd.at[pl.ds(g * GRP, GRP)], s2))
            for cp in cps:
                cp.wait()
            pltpu.sync_copy(rs, hs_h.at[pl.ds(base, c)])
            pltpu.sync_copy(rd, hd_h.at[pl.ds(base, c)])
            return 0
        lax.fori_loop(0, nch, chunk, 0)

    return k(src, dst, hfin)


def _elu(x):
    return jnp.where(x > 0, x, jnp.exp(jnp.minimum(x, 0.0)) - 1.0)


def _softmax_pieces(dp_ref, asp_ref, adp_ref, c_ref, h_ref, mp_ref):
    sa = asp_ref[:, :HEADS] + adp_ref[:, :HEADS]
    ex_ii = jnp.exp(jnp.maximum(sa, 0.2 * sa) - c_ref[...])
    denom = dp_ref[0][:, :HEADS] + dp_ref[1][:, :HEADS] + ex_ii
    rden = 1.0 / (denom + 1e-16)
    pieces = []
    for hd in range(HEADS):
        agg0 = mp_ref[2 * hd, 0] + mp_ref[2 * hd, 1]
        agg1 = mp_ref[2 * hd + 1, 0] + mp_ref[2 * hd + 1, 1]
        aggh = jnp.concatenate([agg0, agg1], axis=1)
        aggh = aggh + h_ref[:, HID * hd:HID * (hd + 1)] * ex_ii[:, hd:hd + 1]
        pieces.append(aggh * rden[:, hd:hd + 1])
    return pieces


def _combine1_body(dp_ref, asp_ref, adp_ref, c_ref, h_ref, mp_ref,
                   b_ref, w2_ref, as2_ref, ad2_ref,
                   h2_ref, a2sp_ref, a2dp_ref, ms_ref, md_ref):
    i = pl.program_id(0)
    pieces = _softmax_pieces(dp_ref, asp_ref, adp_ref, c_ref, h_ref, mp_ref)
    o = _elu(jnp.concatenate(pieces, axis=1) + b_ref[...])
    h2 = jnp.dot(o, w2_ref[...], preferred_element_type=jnp.float32)
    h2_ref[...] = h2
    a_s = jnp.dot(h2, as2_ref[...], preferred_element_type=jnp.float32)
    a_d = jnp.dot(h2, ad2_ref[...], preferred_element_type=jnp.float32)
    bn = a_s.shape[0]
    pad = jnp.zeros((bn, L - HEADS), jnp.float32)
    a2sp_ref[...] = jnp.concatenate([a_s, pad], axis=1)
    a2dp_ref[...] = jnp.concatenate([a_d, pad], axis=1)
    bs = jnp.max(a_s, axis=0, keepdims=True)
    bd = jnp.max(a_d, axis=0, keepdims=True)

    @pl.when(i == 0)
    def _():
        ms_ref[...] = bs
        md_ref[...] = bd

    @pl.when(i != 0)
    def _():
        ms_ref[...] = jnp.maximum(ms_ref[...], bs)
        md_ref[...] = jnp.maximum(md_ref[...], bd)


def _combine1(dp, asp, adp, c, H, mp, b, W2, As2, Ad2, bn=1000):
    n = H.shape[0]
    return pl.pallas_call(
        _combine1_body,
        grid=(n // bn,),
        in_specs=[
            pl.BlockSpec((NC, bn, L), lambda i: (0, i, 0)),
            pl.BlockSpec((bn, L), lambda i: (i, 0)),
            pl.BlockSpec((bn, L), lambda i: (i, 0)),
            pl.BlockSpec((1, HEADS), lambda i: (0, 0)),
            pl.BlockSpec((bn, F), lambda i: (i, 0)),
            pl.BlockSpec((8, NC, bn, L), lambda i: (0, 0, i, 0)),
            pl.BlockSpec((1, F), lambda i: (0, 0)),
            pl.BlockSpec((F, F), lambda i: (0, 0)),
            pl.BlockSpec((F, HEADS), lambda i: (0, 0)),
            pl.BlockSpec((F, HEADS), lambda i: (0, 0)),
        ],
        out_specs=[
            pl.BlockSpec((bn, F), lambda i: (i, 0)),
            pl.BlockSpec((bn, L), lambda i: (i, 0)),
            pl.BlockSpec((bn, L), lambda i: (i, 0)),
            pl.BlockSpec((1, HEADS), lambda i: (0, 0)),
            pl.BlockSpec((1, HEADS), lambda i: (0, 0)),
        ],
        out_shape=[
            jax.ShapeDtypeStruct((n, F), jnp.float32),
            jax.ShapeDtypeStruct((n, L), jnp.float32),
            jax.ShapeDtypeStruct((n, L), jnp.float32),
            jax.ShapeDtypeStruct((1, HEADS), jnp.float32),
            jax.ShapeDtypeStruct((1, HEADS), jnp.float32),
        ],
    )(dp, asp, adp, c, H, mp, b, W2, As2, Ad2)


def _combine2_body(dp_ref, asp_ref, adp_ref, c_ref, h_ref, mp_ref, b_ref,
                   out_ref):
    pieces = _softmax_pieces(dp_ref, asp_ref, adp_ref, c_ref, h_ref, mp_ref)
    acc = pieces[0]
    for p in pieces[1:]:
        acc = acc + p
    out_ref[...] = _elu(acc * (1.0 / HEADS) + b_ref[...])


def _combine2(dp, asp, adp, c, H, mp, b, bn=1000):
    n = H.shape[0]
    return pl.pallas_call(
        _combine2_body,
        grid=(n // bn,),
        in_specs=[
            pl.BlockSpec((NC, bn, L), lambda i: (0, i, 0)),
            pl.BlockSpec((bn, L), lambda i: (i, 0)),
            pl.BlockSpec((bn, L), lambda i: (i, 0)),
            pl.BlockSpec((1, HEADS), lambda i: (0, 0)),
            pl.BlockSpec((bn, F), lambda i: (i, 0)),
            pl.BlockSpec((8, NC, bn, L), lambda i: (0, 0, i, 0)),
            pl.BlockSpec((1, HID), lambda i: (0, 0)),
        ],
        out_specs=pl.BlockSpec((bn, HID), lambda i: (i, 0)),
        out_shape=jax.ShapeDtypeStruct((n, HID), jnp.float32),
    )(dp, asp, adp, c, H, mp, b)


def _decoder_body(hs_ref, hd_ref, ea_ref, w1s_ref, w1d_ref, w1e_ref, b1_ref,
                  w2_ref, b2_ref, w3_ref, b3_ref, out_ref):
    z = jnp.dot(hs_ref[...], w1s_ref[...], preferred_element_type=jnp.float32)
    z += jnp.dot(hd_ref[...], w1d_ref[...], preferred_element_type=jnp.float32)
    z += jnp.dot(ea_ref[...], w1e_ref[...], preferred_element_type=jnp.float32)
    z = jnp.maximum(z + b1_ref[...], 0.0)
    z = jnp.maximum(
        jnp.dot(z, w2_ref[...], preferred_element_type=jnp.float32)
        + b2_ref[...], 0.0)
    out_ref[...] = (
        jnp.dot(z, w3_ref[...], preferred_element_type=jnp.float32)
        + b3_ref[...])


def _decoder(hs, hd, ea, Wd1, bd1, Wd2, bd2, Wd3, bd3, be=8000):
    e = hs.shape[0]
    w1s, w1d, w1e = Wd1[:HID], Wd1[HID:2 * HID], Wd1[2 * HID:]
    return pl.pallas_call(
        _decoder_body,
        grid=(e // be,),
        in_specs=[
            pl.BlockSpec((be, HID), lambda i: (i, 0)),
            pl.BlockSpec((be, HID), lambda i: (i, 0)),
            pl.BlockSpec((be, 5), lambda i: (i, 0)),
            pl.BlockSpec((HID, 2 * HID), lambda i: (0, 0)),
            pl.BlockSpec((HID, 2 * HID), lambda i: (0, 0)),
            pl.BlockSpec((5, 2 * HID), lambda i: (0, 0)),
            pl.BlockSpec((1, 2 * HID), lambda i: (0, 0)),
            pl.BlockSpec((2 * HID, HID), lambda i: (0, 0)),
            pl.BlockSpec((1, HID), lambda i: (0, 0)),
            pl.BlockSpec((HID, 4), lambda i: (0, 0)),
            pl.BlockSpec((1, 4), lambda i: (0, 0)),
        ],
        out_specs=pl.BlockSpec((be, 4), lambda i: (i, 0)),
        out_shape=jax.ShapeDtypeStruct((e, 4), jnp.float32),
    )(hs, hd, ea, w1s, w1d, w1e, bd1[None], Wd2, bd2[None], Wd3, bd3[None])


def _block_diag_att(att):
    m = jnp.zeros((HEADS, HID, HEADS), dtype=att.dtype)
    m = m.at[jnp.arange(HEADS), :, jnp.arange(HEADS)].set(att[0])
    return m.reshape(F, HEADS)


def _shift(ms, md):
    sa = ms + md
    c = jnp.maximum(sa, 0.2 * sa)
    cvec = jnp.tile(c[0], HEADS)
    return c, cvec


def kernel(x, edge_index, edge_attr, u, W1, att1_src, att1_dst, b1,
           W2, att2_src, att2_dst, b2, Wd1, bd1, Wd2, bd2, Wd3, bd3):
    n = x.shape[0]
    e = edge_index.shape[1]
    src = edge_index[0]
    dst = edge_index[1]
    del e
    zeros16 = jnp.zeros((n, L), jnp.float32)

    u_node = jnp.broadcast_to(u, (n, u.shape[1]))
    h0 = jnp.concatenate([x, u_node], axis=-1)

    H1, a1sp, a1dp, ms1, md1 = _proj(
        h0, W1, _block_diag_att(att1_src), _block_diag_att(att1_dst))
    c1, c1vec = _shift(ms1, md1)
    exT1, dp1 = _sc_pass_a(src, dst, a1sp, a1dp, c1vec, zeros16)
    mp1 = _sc_pass_b(src, dst, exT1, H1.reshape(8 * n, L), zeros16)
    H2, a2sp, a2dp, ms2, md2 = _combine1(
        dp1, a1sp, a1dp, c1, H1, mp1, b1[None], W2,
        _block_diag_att(att2_src), _block_diag_att(att2_dst))

    c2, c2vec = _shift(ms2, md2)
    exT2, dp2 = _sc_pass_a(src, dst, a2sp, a2dp, c2vec, zeros16)
    mp2 = _sc_pass_b(src, dst, exT2, H2.reshape(8 * n, L), zeros16)
    hfin = _combine2(dp2, a2sp, a2dp, c2, H2, mp2, b2[None])

    hs, hd = _sc_pass_d(src, dst, hfin)
    return _decoder(hs, hd, edge_attr, Wd1, bd1, Wd2, bd2, Wd3, bd3)


# speedup vs baseline: 24.6224x; 1.0395x over previous
import functools

import jax
import jax.numpy as jnp
from jax import lax
from jax.experimental import pallas as pl
from jax.experimental.pallas import tpu as pltpu
from jax.experimental.pallas import tpu_sc as plsc

HEADS = 4
HID = 32
F = HEADS * HID
L = 16
NC = 2
NS = 16
NW = NC * NS
GRP = 80


def _mesh():
    return plsc.VectorSubcoreMesh(
        core_axis_name="c", subcore_axis_name="s",
        num_cores=NC, num_subcores=NS)


def _proj_body(h0_ref, w_ref, as_ref, ad_ref,
               h_ref, asp_ref, adp_ref, ms_ref, md_ref):
    i = pl.program_id(0)
    h = jnp.dot(h0_ref[...], w_ref[...], preferred_element_type=jnp.float32)
    h_ref[...] = h
    a_s = jnp.dot(h, as_ref[...], preferred_element_type=jnp.float32)
    a_d = jnp.dot(h, ad_ref[...], preferred_element_type=jnp.float32)
    bn = a_s.shape[0]
    pad = jnp.zeros((bn, L - HEADS), jnp.float32)
    asp_ref[...] = jnp.concatenate([a_s, pad], axis=1)
    adp_ref[...] = jnp.concatenate([a_d, pad], axis=1)
    bs = jnp.max(a_s, axis=0, keepdims=True)
    bd = jnp.max(a_d, axis=0, keepdims=True)

    @pl.when(i == 0)
    def _():
        ms_ref[...] = bs
        md_ref[...] = bd

    @pl.when(i != 0)
    def _():
        ms_ref[...] = jnp.maximum(ms_ref[...], bs)
        md_ref[...] = jnp.maximum(md_ref[...], bd)


def _proj(h0, W, As, Ad, bn=1000):
    n, k = h0.shape
    return pl.pallas_call(
        _proj_body,
        grid=(n // bn,),
        in_specs=[
            pl.BlockSpec((bn, k), lambda i: (i, 0)),
            pl.BlockSpec((k, F), lambda i: (0, 0)),
            pl.BlockSpec((F, HEADS), lambda i: (0, 0)),
            pl.BlockSpec((F, HEADS), lambda i: (0, 0)),
        ],
        out_specs=[
            pl.BlockSpec((bn, F), lambda i: (i, 0)),
            pl.BlockSpec((bn, L), lambda i: (i, 0)),
            pl.BlockSpec((bn, L), lambda i: (i, 0)),
            pl.BlockSpec((1, HEADS), lambda i: (0, 0)),
            pl.BlockSpec((1, HEADS), lambda i: (0, 0)),
        ],
        out_shape=[
            jax.ShapeDtypeStruct((n, F), jnp.float32),
            jax.ShapeDtypeStruct((n, L), jnp.float32),
            jax.ShapeDtypeStruct((n, L), jnp.float32),
            jax.ShapeDtypeStruct((1, HEADS), jnp.float32),
            jax.ShapeDtypeStruct((1, HEADS), jnp.float32),
        ],
    )(h0, W, As, Ad)


def _stripe_zero(z_h, acc, sid, s0, tail):
    pltpu.sync_copy(z_h.at[pl.ds(sid * s0, s0)], acc.at[pl.ds(sid * s0, s0)])

    @pl.when(sid == NS - 1)
    def _():
        pltpu.sync_copy(z_h.at[pl.ds(NS * s0, tail)],
                        acc.at[pl.ds(NS * s0, tail)])


def _stripe_drain(acc, dst_ref, sid, s0, tail):
    pltpu.sync_copy(acc.at[pl.ds(sid * s0, s0)],
                    dst_ref.at[pl.ds(sid * s0, s0)])

    @pl.when(sid == NS - 1)
    def _():
        pltpu.sync_copy(acc.at[pl.ds(NS * s0, tail)],
                        dst_ref.at[pl.ds(NS * s0, tail)])


def _load_dst_groups(dst_h, dstb, base, ng, sem):
    def fire(g, _):
        pltpu.async_copy(dst_h.at[pl.ds(base + g * GRP, GRP)],
                         dstb.at[g], sem)
        return 0
    lax.fori_loop(0, ng, fire, 0)

    def drain(g, _):
        pltpu.make_async_copy(dst_h.at[pl.ds(base, GRP)],
                              dstb.at[0], sem).wait()
        return 0
    lax.fori_loop(0, ng, drain, 0)


def _range_idx(dstb, dstb2, lo, n2, ng):
    kk = GRP // L

    @plsc.parallel_loop(0, ng * kk, 1, unroll=4)
    def _(q):
        g = q // kk
        k = q % kk
        dv = dstb[g, pl.ds(k * L, L)]
        ok = (dv >= lo) & (dv < lo + n2)
        dstb2[g, pl.ds(k * L, L)] = jnp.where(ok, dv - lo, n2)


def _fire_gathers(table_h, idx_ref, rows_ref, ng, sem):
    def fire(g, _):
        pltpu.async_copy(table_h.at[idx_ref.at[pl.ds(g * GRP, GRP)]],
                         rows_ref.at[pl.ds(g * GRP, GRP)], sem)
        return 0
    lax.fori_loop(0, ng, fire, 0)


def _drain_gathers(table_h, idx_ref, rows_ref, ng, sem):
    def drain(g, _):
        pltpu.make_async_copy(table_h.at[idx_ref.at[pl.ds(0, GRP)]],
                              rows_ref.at[pl.ds(0, GRP)], sem).wait()
        return 0
    lax.fori_loop(0, ng, drain, 0)


def _scatter_add_groups(rows_ref, acc, idx2_ref, ng, sem):
    def fire(g, _):
        pltpu.async_copy(rows_ref.at[pl.ds(g * GRP, GRP)],
                         acc.at[idx2_ref.at[g]], sem, add=True)
        return 0
    lax.fori_loop(0, ng, fire, 0)

    def drain(g, _):
        pltpu.make_async_copy(rows_ref.at[pl.ds(0, GRP)],
                              acc.at[idx2_ref.at[0]], sem).wait()
        return 0
    lax.fori_loop(0, ng, drain, 0)


def _sc_pass_a(src, dst, asp, adp, cvec, zeros16):
    e = src.shape[0]
    n = asp.shape[0]
    n2 = n // 2
    ew = e // NW
    c = 400
    ng = c // GRP
    nch = ew // c
    s0 = (n2 // NS) & ~7
    tail = n2 - NS * s0

    @functools.partial(
        pl.kernel,
        out_type=[
            jax.ShapeDtypeStruct((HEADS * e,), jnp.float32),
            jax.ShapeDtypeStruct((NC, n, L), jnp.float32),
        ],
        mesh=_mesh(),
        compiler_params=pltpu.CompilerParams(use_tc_tiling_on_sc=False),
        scratch_types=[
            pltpu.VMEM((c,), jnp.int32),
            pltpu.VMEM((ng, GRP), jnp.int32),
            pltpu.VMEM((ng, GRP), jnp.int32),
            pltpu.VMEM((c, L), jnp.float32),
            pltpu.VMEM((c, L), jnp.float32),
            pltpu.VMEM((c, L), jnp.float32),
            pltpu.VMEM((HEADS * c,), jnp.float32),
            pltpu.VMEM((L,), jnp.float32),
            pltpu.VMEM_SHARED((n2 + 8, L), jnp.float32),
            pltpu.SemaphoreType.DMA,
            pltpu.SemaphoreType.DMA,
        ],
    )
    def k(src_h, dst_h, asp_h, adp_h, cvec_h, z_h, exT_h, dp_h,
          srcb, dstb, dstb2, asb, adb, exb, exTb, cb, acc, s1, s2):
        cid = lax.axis_index("c")
        sid = lax.axis_index("s")
        wid = sid * NC + cid
        pltpu.sync_copy(cvec_h, cb)
        cv = cb[...]
        iota = lax.iota(jnp.int32, L)
        iotam4 = iota % HEADS
        m4 = iota < 4
        m8 = iota < 8
        m12 = iota < 12

        for r in range(2):
            lo = r * n2
            _stripe_zero(z_h, acc, sid, s0, tail + 8)
            plsc.subcore_barrier()

            def chunk(i, _):
                base = wid * ew + i * c
                pltpu.sync_copy(src_h.at[pl.ds(base, c)], srcb)
                _load_dst_groups(dst_h, dstb, base, ng, s2)
                _fire_gathers(asp_h, srcb, asb, ng, s1)

                def fire_ad(g, _):
                    pltpu.async_copy(adp_h.at[dstb.at[g]],
                                     adb.at[pl.ds(g * GRP, GRP)], s2)
                    return 0
                lax.fori_loop(0, ng, fire_ad, 0)
                _range_idx(dstb, dstb2, lo, n2, ng)
                _drain_gathers(asp_h, srcb, asb, ng, s1)

                def drain_ad(g, _):
                    pltpu.make_async_copy(adp_h.at[dstb.at[0]],
                                          adb.at[pl.ds(0, GRP)], s2).wait()
                    return 0
                lax.fori_loop(0, ng, drain_ad, 0)

                @plsc.parallel_loop(0, c, 4, unroll=4)
                def _(j0):
                    gs = []
                    for t in range(4):
                        j = j0 + t
                        s = asb[j] + adb[j]
                        v = jnp.exp(jnp.maximum(s, 0.2 * s) - cv)
                        exb[j] = v
                        if r == 0:
                            gs.append(
                                v.at[iotam4].get(mode="promise_in_bounds"))
                    if r == 0:
                        q = jnp.where(m4, gs[0],
                                      jnp.where(m8, gs[1],
                                                jnp.where(m12, gs[2], gs[3])))
                        exTb[pl.ds(j0 * HEADS, L)] = q

                if r == 0:
                    pltpu.sync_copy(exTb,
                                    exT_h.at[pl.ds(base * HEADS, c * HEADS)])
                _scatter_add_groups(exb, acc, dstb2, ng, s1)
                return 0
            lax.fori_loop(0, nch, chunk, 0)
            plsc.subcore_barrier()
            _stripe_drain(acc, dp_h.at[cid].at[pl.ds(lo, n2)], sid, s0, tail)
            plsc.subcore_barrier()

    return k(src, dst, asp, adp, cvec, zeros16)


def _sc_pass_b(src, dst, exT, h8, zeros16):
    e = src.shape[0]
    n = h8.shape[0] // 8
    n2 = n // 2
    ew = e // NW
    c = 2000
    ng = c // GRP
    nch = ew // c
    s0 = (n2 // NS) & ~7
    tail = n2 - NS * s0

    @functools.partial(
        pl.kernel,
        out_type=jax.ShapeDtypeStruct((8, NC, n, L), jnp.float32),
        mesh=_mesh(),
        compiler_params=pltpu.CompilerParams(use_tc_tiling_on_sc=False),
        scratch_types=[
            pltpu.VMEM((c,), jnp.int32),
            pltpu.VMEM((ng, GRP), jnp.int32),
            pltpu.VMEM((ng, GRP), jnp.int32),
            pltpu.VMEM((c,), jnp.int32),
            pltpu.VMEM((HEADS * c,), jnp.float32),
            pltpu.VMEM((c, L), jnp.float32),
            pltpu.VMEM_SHARED((n2 + 8, L), jnp.float32),
            pltpu.SemaphoreType.DMA,
            pltpu.SemaphoreType.DMA,
        ],
    )
    def k(src_h, dst_h, exT_h, h8_h, z_h, out_h,
          srcb, dstb, dstb2, idxb, exw, rows, acc, s1, s2):
        cid = lax.axis_index("c")
        sid = lax.axis_index("s")
        wid = sid * NC + cid
        for hd in range(HEADS):
            def sweep(q, _):
                f = q // 2
                r = q % 2
                slot = hd * 2 + f
                lo = r * n2
                _stripe_zero(z_h, acc, sid, s0, tail + 8)
                plsc.subcore_barrier()

                def chunk(i, _):
                    base = wid * ew + i * c
                    pltpu.sync_copy(src_h.at[pl.ds(base, c)], srcb)
                    _load_dst_groups(dst_h, dstb, base, ng, s2)
                    pltpu.sync_copy(
                        exT_h.at[pl.ds(base * HEADS, c * HEADS)], exw)

                    @plsc.parallel_loop(0, c, L, unroll=4)
                    def _(j0):
                        v = srcb[pl.ds(j0, L)]
                        idxb[pl.ds(j0, L)] = v * 8 + slot
                    _fire_gathers(h8_h, idxb, rows, ng, s1)
                    _range_idx(dstb, dstb2, lo, n2, ng)
                    _drain_gathers(h8_h, idxb, rows, ng, s1)

                    @plsc.parallel_loop(0, c, 4, unroll=8)
                    def _(j0):
                        ex16 = exw[pl.ds(j0 * HEADS, L)]
                        for t in range(4):
                            rows[j0 + t] = rows[j0 + t] * ex16[t * HEADS + hd]
                    _scatter_add_groups(rows, acc, dstb2, ng, s1)
                    return 0
                lax.fori_loop(0, nch, chunk, 0)
                plsc.subcore_barrier()
                _stripe_drain(acc, out_h.at[slot, cid].at[pl.ds(lo, n2)],
                              sid, s0, tail)
                plsc.subcore_barrier()
                return 0
            lax.fori_loop(0, 4, sweep, 0)

    return k(src, dst, exT, h8, zeros16)


def _sc_pass_d(src, dst, hfin):
    e = src.shape[0]
    ew = e // NW
    c = 400
    ng = c // GRP
    nch = ew // c

    @functools.partial(
        pl.kernel,
        out_type=[
            jax.ShapeDtypeStruct((e, HID), jnp.float32),
            jax.ShapeDtypeStruct((e, HID), jnp.float32),
        ],
        mesh=_mesh(),
        compiler_params=pltpu.CompilerParams(
            use_tc_tiling_on_sc=False, internal_scratch_in_bytes=1 << 16),
        scratch_types=[
            pltpu.VMEM((c,), jnp.int32),
            pltpu.VMEM((c,), jnp.int32),
            pltpu.VMEM((c, HID), jnp.float32),
            pltpu.VMEM((c, HID), jnp.float32),
            pltpu.SemaphoreType.DMA,
            pltpu.SemaphoreType.DMA,
        ],
    )
    def k(src_h, dst_h, hf_h, hs_h, hd_h, srcb, dstb, rs, rd, s1, s2):
        cid = lax.axis_index("c")
        sid = lax.axis_index("s")
        wid = sid * NC + cid

        def chunk(i, _):
            base = wid * ew + i * c
            pltpu.sync_copy(src_h.at[pl.ds(base, c)], srcb)
            pltpu.sync_copy(dst_h.at[pl.ds(base, c)], dstb)
            cps = []
            for g in range(ng):
                cps.append(pltpu.async_copy(
                    hf_h.at[srcb.at[pl.ds(g * GRP, GRP)]],
                    rs.at[pl.ds(g * GRP, GRP)], s1))
                cps.append(pltpu.async_copy(
                    hf_h.at[dstb.at[pl.ds(g * GRP, GRP)]],
                    rd.at[pl.ds(g * GRP, GRP)], s2))
            for cp in cps:
                cp.wait()
            pltpu.sync_copy(rs, hs_h.at[pl.ds(base, c)])
            pltpu.sync_copy(rd, hd_h.at[pl.ds(base, c)])
            return 0
        lax.fori_loop(0, nch, chunk, 0)

    return k(src, dst, hfin)


def _elu(x):
    return jnp.where(x > 0, x, jnp.exp(jnp.minimum(x, 0.0)) - 1.0)


def _softmax_pieces(dp_ref, asp_ref, adp_ref, c_ref, h_ref, mp_ref):
    sa = asp_ref[:, :HEADS] + adp_ref[:, :HEADS]
    ex_ii = jnp.exp(jnp.maximum(sa, 0.2 * sa) - c_ref[...])
    denom = dp_ref[0][:, :HEADS] + dp_ref[1][:, :HEADS] + ex_ii
    rden = 1.0 / (denom + 1e-16)
    pieces = []
    for hd in range(HEADS):
        agg0 = mp_ref[2 * hd, 0] + mp_ref[2 * hd, 1]
        agg1 = mp_ref[2 * hd + 1, 0] + mp_ref[2 * hd + 1, 1]
        aggh = jnp.concatenate([agg0, agg1], axis=1)
        aggh = aggh + h_ref[:, HID * hd:HID * (hd + 1)] * ex_ii[:, hd:hd + 1]
        pieces.append(aggh * rden[:, hd:hd + 1])
    return pieces


def _combine1_body(dp_ref, asp_ref, adp_ref, c_ref, h_ref, mp_ref,
                   b_ref, w2_ref, as2_ref, ad2_ref,
                   h2_ref, a2sp_ref, a2dp_ref, ms_ref, md_ref):
    i = pl.program_id(0)
    pieces = _softmax_pieces(dp_ref, asp_ref, adp_ref, c_ref, h_ref, mp_ref)
    o = _elu(jnp.concatenate(pieces, axis=1) + b_ref[...])
    h2 = jnp.dot(o, w2_ref[...], preferred_element_type=jnp.float32)
    h2_ref[...] = h2
    a_s = jnp.dot(h2, as2_ref[...], preferred_element_type=jnp.float32)
    a_d = jnp.dot(h2, ad2_ref[...], preferred_element_type=jnp.float32)
    bn = a_s.shape[0]
    pad = jnp.zeros((bn, L - HEADS), jnp.float32)
    a2sp_ref[...] = jnp.concatenate([a_s, pad], axis=1)
    a2dp_ref[...] = jnp.concatenate([a_d, pad], axis=1)
    bs = jnp.max(a_s, axis=0, keepdims=True)
    bd = jnp.max(a_d, axis=0, keepdims=True)

    @pl.when(i == 0)
    def _():
        ms_ref[...] = bs
        md_ref[...] = bd

    @pl.when(i != 0)
    def _():
        ms_ref[...] = jnp.maximum(ms_ref[...], bs)
        md_ref[...] = jnp.maximum(md_ref[...], bd)


def _combine1(dp, asp, adp, c, H, mp, b, W2, As2, Ad2, bn=1000):
    n = H.shape[0]
    return pl.pallas_call(
        _combine1_body,
        grid=(n // bn,),
        in_specs=[
            pl.BlockSpec((NC, bn, L), lambda i: (0, i, 0)),
            pl.BlockSpec((bn, L), lambda i: (i, 0)),
            pl.BlockSpec((bn, L), lambda i: (i, 0)),
            pl.BlockSpec((1, HEADS), lambda i: (0, 0)),
            pl.BlockSpec((bn, F), lambda i: (i, 0)),
            pl.BlockSpec((8, NC, bn, L), lambda i: (0, 0, i, 0)),
            pl.BlockSpec((1, F), lambda i: (0, 0)),
            pl.BlockSpec((F, F), lambda i: (0, 0)),
            pl.BlockSpec((F, HEADS), lambda i: (0, 0)),
            pl.BlockSpec((F, HEADS), lambda i: (0, 0)),
        ],
        out_specs=[
            pl.BlockSpec((bn, F), lambda i: (i, 0)),
            pl.BlockSpec((bn, L), lambda i: (i, 0)),
            pl.BlockSpec((bn, L), lambda i: (i, 0)),
            pl.BlockSpec((1, HEADS), lambda i: (0, 0)),
            pl.BlockSpec((1, HEADS), lambda i: (0, 0)),
        ],
        out_shape=[
            jax.ShapeDtypeStruct((n, F), jnp.float32),
            jax.ShapeDtypeStruct((n, L), jnp.float32),
            jax.ShapeDtypeStruct((n, L), jnp.float32),
            jax.ShapeDtypeStruct((1, HEADS), jnp.float32),
            jax.ShapeDtypeStruct((1, HEADS), jnp.float32),
        ],
    )(dp, asp, adp, c, H, mp, b, W2, As2, Ad2)


def _combine2_body(dp_ref, asp_ref, adp_ref, c_ref, h_ref, mp_ref, b_ref,
                   out_ref):
    pieces = _softmax_pieces(dp_ref, asp_ref, adp_ref, c_ref, h_ref, mp_ref)
    acc = pieces[0]
    for p in pieces[1:]:
        acc = acc + p
    out_ref[...] = _elu(acc * (1.0 / HEADS) + b_ref[...])


def _combine2(dp, asp, adp, c, H, mp, b, bn=1000):
    n = H.shape[0]
    return pl.pallas_call(
        _combine2_body,
        grid=(n // bn,),
        in_specs=[
            pl.BlockSpec((NC, bn, L), lambda i: (0, i, 0)),
            pl.BlockSpec((bn, L), lambda i: (i, 0)),
            pl.BlockSpec((bn, L), lambda i: (i, 0)),
            pl.BlockSpec((1, HEADS), lambda i: (0, 0)),
            pl.BlockSpec((bn, F), lambda i: (i, 0)),
            pl.BlockSpec((8, NC, bn, L), lambda i: (0, 0, i, 0)),
            pl.BlockSpec((1, HID), lambda i: (0, 0)),
        ],
        out_specs=pl.BlockSpec((bn, HID), lambda i: (i, 0)),
        out_shape=jax.ShapeDtypeStruct((n, HID), jnp.float32),
    )(dp, asp, adp, c, H, mp, b)


def _decoder_body(hs_ref, hd_ref, ea_ref, w1s_ref, w1d_ref, w1e_ref, b1_ref,
                  w2_ref, b2_ref, w3_ref, b3_ref, out_ref):
    z = jnp.dot(hs_ref[...], w1s_ref[...], preferred_element_type=jnp.float32)
    z += jnp.dot(hd_ref[...], w1d_ref[...], preferred_element_type=jnp.float32)
    z += jnp.dot(ea_ref[...], w1e_ref[...], preferred_element_type=jnp.float32)
    z = jnp.maximum(z + b1_ref[...], 0.0)
    z = jnp.maximum(
        jnp.dot(z, w2_ref[...], preferred_element_type=jnp.float32)
        + b2_ref[...], 0.0)
    out_ref[...] = (
        jnp.dot(z, w3_ref[...], preferred_element_type=jnp.float32)
        + b3_ref[...])


def _decoder(hs, hd, ea, Wd1, bd1, Wd2, bd2, Wd3, bd3, be=8000):
    e = hs.shape[0]
    w1s, w1d, w1e = Wd1[:HID], Wd1[HID:2 * HID], Wd1[2 * HID:]
    return pl.pallas_call(
        _decoder_body,
        grid=(e // be,),
        in_specs=[
            pl.BlockSpec((be, HID), lambda i: (i, 0)),
            pl.BlockSpec((be, HID), lambda i: (i, 0)),
            pl.BlockSpec((be, 5), lambda i: (i, 0)),
            pl.BlockSpec((HID, 2 * HID), lambda i: (0, 0)),
            pl.BlockSpec((HID, 2 * HID), lambda i: (0, 0)),
            pl.BlockSpec((5, 2 * HID), lambda i: (0, 0)),
            pl.BlockSpec((1, 2 * HID), lambda i: (0, 0)),
            pl.BlockSpec((2 * HID, HID), lambda i: (0, 0)),
            pl.BlockSpec((1, HID), lambda i: (0, 0)),
            pl.BlockSpec((HID, 4), lambda i: (0, 0)),
            pl.BlockSpec((1, 4), lambda i: (0, 0)),
        ],
        out_specs=pl.BlockSpec((be, 4), lambda i: (i, 0)),
        out_shape=jax.ShapeDtypeStruct((e, 4), jnp.float32),
    )(hs, hd, ea, w1s, w1d, w1e, bd1[None], Wd2, bd2[None], Wd3, bd3[None])


def _block_diag_att(att):
    m = jnp.zeros((HEADS, HID, HEADS), dtype=att.dtype)
    m = m.at[jnp.arange(HEADS), :, jnp.arange(HEADS)].set(att[0])
    return m.reshape(F, HEADS)


def _shift(ms, md):
    sa = ms + md
    c = jnp.maximum(sa, 0.2 * sa)
    cvec = jnp.tile(c[0], HEADS)
    return c, cvec


def kernel(x, edge_index, edge_attr, u, W1, att1_src, att1_dst, b1,
           W2, att2_src, att2_dst, b2, Wd1, bd1, Wd2, bd2, Wd3, bd3):
    n = x.shape[0]
    e = edge_index.shape[1]
    src = edge_index[0]
    dst = edge_index[1]
    del e
    zeros16 = jnp.zeros((n, L), jnp.float32)

    u_node = jnp.broadcast_to(u, (n, u.shape[1]))
    h0 = jnp.concatenate([x, u_node], axis=-1)

    H1, a1sp, a1dp, ms1, md1 = _proj(
        h0, W1, _block_diag_att(att1_src), _block_diag_att(att1_dst))
    c1, c1vec = _shift(ms1, md1)
    exT1, dp1 = _sc_pass_a(src, dst, a1sp, a1dp, c1vec, zeros16)
    mp1 = _sc_pass_b(src, dst, exT1, H1.reshape(8 * n, L), zeros16)
    H2, a2sp, a2dp, ms2, md2 = _combine1(
        dp1, a1sp, a1dp, c1, H1, mp1, b1[None], W2,
        _block_diag_att(att2_src), _block_diag_att(att2_dst))

    c2, c2vec = _shift(ms2, md2)
    exT2, dp2 = _sc_pass_a(src, dst, a2sp, a2dp, c2vec, zeros16)
    mp2 = _sc_pass_b(src, dst, exT2, H2.reshape(8 * n, L), zeros16)
    hfin = _combine2(dp2, a2sp, a2dp, c2, H2, mp2, b2[None])

    hs, hd = _sc_pass_d(src, dst, hfin)
    return _decoder(hs, hd, edge_attr, Wd1, bd1, Wd2, bd2, Wd3, bd3)


# speedup vs baseline: 36.9460x; 1.5005x over previous
import functools

import jax
import jax.numpy as jnp
from jax import lax
from jax.experimental import pallas as pl
from jax.experimental.pallas import tpu as pltpu
from jax.experimental.pallas import tpu_sc as plsc

HEADS = 4
HID = 32
F = HEADS * HID
L = 16
NC = 2
NS = 16
NW = NC * NS
GRP = 80


def _mesh():
    return plsc.VectorSubcoreMesh(
        core_axis_name="c", subcore_axis_name="s",
        num_cores=NC, num_subcores=NS)


def _proj_body(h0_ref, w_ref, as_ref, ad_ref,
               h_ref, asp_ref, adp_ref, ms_ref, md_ref):
    i = pl.program_id(0)
    h = jnp.dot(h0_ref[...], w_ref[...], preferred_element_type=jnp.float32)
    h_ref[...] = h
    a_s = jnp.dot(h, as_ref[...], preferred_element_type=jnp.float32)
    a_d = jnp.dot(h, ad_ref[...], preferred_element_type=jnp.float32)
    bn = a_s.shape[0]
    pad = jnp.zeros((bn, L - HEADS), jnp.float32)
    asp_ref[...] = jnp.concatenate([a_s, pad], axis=1)
    adp_ref[...] = jnp.concatenate([a_d, pad], axis=1)
    bs = jnp.max(a_s, axis=0, keepdims=True)
    bd = jnp.max(a_d, axis=0, keepdims=True)

    @pl.when(i == 0)
    def _():
        ms_ref[...] = bs
        md_ref[...] = bd

    @pl.when(i != 0)
    def _():
        ms_ref[...] = jnp.maximum(ms_ref[...], bs)
        md_ref[...] = jnp.maximum(md_ref[...], bd)


def _proj(h0, W, As, Ad, bn=1000):
    n, k = h0.shape
    return pl.pallas_call(
        _proj_body,
        grid=(n // bn,),
        in_specs=[
            pl.BlockSpec((bn, k), lambda i: (i, 0)),
            pl.BlockSpec((k, F), lambda i: (0, 0)),
            pl.BlockSpec((F, HEADS), lambda i: (0, 0)),
            pl.BlockSpec((F, HEADS), lambda i: (0, 0)),
        ],
        out_specs=[
            pl.BlockSpec((bn, F), lambda i: (i, 0)),
            pl.BlockSpec((bn, L), lambda i: (i, 0)),
            pl.BlockSpec((bn, L), lambda i: (i, 0)),
            pl.BlockSpec((1, HEADS), lambda i: (0, 0)),
            pl.BlockSpec((1, HEADS), lambda i: (0, 0)),
        ],
        out_shape=[
            jax.ShapeDtypeStruct((n, F), jnp.float32),
            jax.ShapeDtypeStruct((n, L), jnp.float32),
            jax.ShapeDtypeStruct((n, L), jnp.float32),
            jax.ShapeDtypeStruct((1, HEADS), jnp.float32),
            jax.ShapeDtypeStruct((1, HEADS), jnp.float32),
        ],
    )(h0, W, As, Ad)


def _stripe_zero(z_h, acc, sid, s0, tail):
    pltpu.sync_copy(z_h.at[pl.ds(sid * s0, s0)], acc.at[pl.ds(sid * s0, s0)])

    @pl.when(sid == NS - 1)
    def _():
        pltpu.sync_copy(z_h.at[pl.ds(NS * s0, tail)],
                        acc.at[pl.ds(NS * s0, tail)])


def _stripe_drain(acc, dst_ref, sid, s0, tail):
    pltpu.sync_copy(acc.at[pl.ds(sid * s0, s0)],
                    dst_ref.at[pl.ds(sid * s0, s0)])

    @pl.when(sid == NS - 1)
    def _():
        pltpu.sync_copy(acc.at[pl.ds(NS * s0, tail)],
                        dst_ref.at[pl.ds(NS * s0, tail)])


def _load_dst_groups(dst_h, dstb, base, ng, sem):
    def fire(g, _):
        pltpu.async_copy(dst_h.at[pl.ds(base + g * GRP, GRP)],
                         dstb.at[g], sem)
        return 0
    lax.fori_loop(0, ng, fire, 0)

    def drain(g, _):
        pltpu.make_async_copy(dst_h.at[pl.ds(base, GRP)],
                              dstb.at[0], sem).wait()
        return 0
    lax.fori_loop(0, ng, drain, 0)


def _range_idx(dstb, dstb2, lo, n2, ng):
    kk = GRP // L

    @plsc.parallel_loop(0, ng * kk, 1, unroll=4)
    def _(q):
        g = q // kk
        k = q % kk
        dv = dstb[g, pl.ds(k * L, L)]
        ok = (dv >= lo) & (dv < lo + n2)
        dstb2[g, pl.ds(k * L, L)] = jnp.where(ok, dv - lo, n2)


def _fire_gathers(table_h, idx_ref, rows_ref, ng, sem):
    def fire(g, _):
        pltpu.async_copy(table_h.at[idx_ref.at[pl.ds(g * GRP, GRP)]],
                         rows_ref.at[pl.ds(g * GRP, GRP)], sem)
        return 0
    lax.fori_loop(0, ng, fire, 0)


def _drain_gathers(table_h, idx_ref, rows_ref, ng, sem):
    def drain(g, _):
        pltpu.make_async_copy(table_h.at[idx_ref.at[pl.ds(0, GRP)]],
                              rows_ref.at[pl.ds(0, GRP)], sem).wait()
        return 0
    lax.fori_loop(0, ng, drain, 0)


def _scatter_add_groups(rows_ref, acc, idx2_ref, ng, sem):
    def fire(g, _):
        pltpu.async_copy(rows_ref.at[pl.ds(g * GRP, GRP)],
                         acc.at[idx2_ref.at[g]], sem, add=True)
        return 0
    lax.fori_loop(0, ng, fire, 0)

    def drain(g, _):
        pltpu.make_async_copy(rows_ref.at[pl.ds(0, GRP)],
                              acc.at[idx2_ref.at[0]], sem).wait()
        return 0
    lax.fori_loop(0, ng, drain, 0)


def _sc_pass_a(src, dst, asp, adp, cvec, zeros16):
    e = src.shape[0]
    n = asp.shape[0]
    n2 = n // 2
    ew = e // NW
    c = 400
    ng = c // GRP
    nch = ew // c
    s0 = (n2 // NS) & ~7
    tail = n2 - NS * s0

    @functools.partial(
        pl.kernel,
        out_type=[
            jax.ShapeDtypeStruct((HEADS * e,), jnp.float32),
            jax.ShapeDtypeStruct((NC, n, L), jnp.float32),
        ],
        mesh=_mesh(),
        compiler_params=pltpu.CompilerParams(use_tc_tiling_on_sc=False),
        scratch_types=[
            pltpu.VMEM((c,), jnp.int32),
            pltpu.VMEM((ng, GRP), jnp.int32),
            pltpu.VMEM((ng, GRP), jnp.int32),
            pltpu.VMEM((c, L), jnp.float32),
            pltpu.VMEM((c, L), jnp.float32),
            pltpu.VMEM((c, L), jnp.float32),
            pltpu.VMEM((HEADS * c,), jnp.float32),
            pltpu.VMEM((L,), jnp.float32),
            pltpu.VMEM_SHARED((n2 + 8, L), jnp.float32),
            pltpu.SemaphoreType.DMA,
            pltpu.SemaphoreType.DMA,
        ],
    )
    def k(src_h, dst_h, asp_h, adp_h, cvec_h, z_h, exT_h, dp_h,
          srcb, dstb, dstb2, asb, adb, exb, exTb, cb, acc, s1, s2):
        cid = lax.axis_index("c")
        sid = lax.axis_index("s")
        wid = sid * NC + cid
        pltpu.sync_copy(cvec_h, cb)
        cv = cb[...]
        iota = lax.iota(jnp.int32, L)
        iotam4 = iota % HEADS
        m4 = iota < 4
        m8 = iota < 8
        m12 = iota < 12

        for r in range(2):
            lo = r * n2
            _stripe_zero(z_h, acc, sid, s0, tail + 8)
            plsc.subcore_barrier()

            def chunk(i, _):
                base = wid * ew + i * c
                pltpu.sync_copy(src_h.at[pl.ds(base, c)], srcb)
                _load_dst_groups(dst_h, dstb, base, ng, s2)
                _fire_gathers(asp_h, srcb, asb, ng, s1)

                def fire_ad(g, _):
                    pltpu.async_copy(adp_h.at[dstb.at[g]],
                                     adb.at[pl.ds(g * GRP, GRP)], s2)
                    return 0
                lax.fori_loop(0, ng, fire_ad, 0)
                _range_idx(dstb, dstb2, lo, n2, ng)
                _drain_gathers(asp_h, srcb, asb, ng, s1)

                def drain_ad(g, _):
                    pltpu.make_async_copy(adp_h.at[dstb.at[0]],
                                          adb.at[pl.ds(0, GRP)], s2).wait()
                    return 0
                lax.fori_loop(0, ng, drain_ad, 0)

                @plsc.parallel_loop(0, c, 4, unroll=4)
                def _(j0):
                    gs = []
                    for t in range(4):
                        j = j0 + t
                        s = asb[j] + adb[j]
                        v = jnp.exp(jnp.maximum(s, 0.2 * s) - cv)
                        exb[j] = v
                        if r == 0:
                            gs.append(
                                v.at[iotam4].get(mode="promise_in_bounds"))
                    if r == 0:
                        q = jnp.where(m4, gs[0],
                                      jnp.where(m8, gs[1],
                                                jnp.where(m12, gs[2], gs[3])))
                        exTb[pl.ds(j0 * HEADS, L)] = q

                if r == 0:
                    pltpu.sync_copy(exTb,
                                    exT_h.at[pl.ds(base * HEADS, c * HEADS)])
                _scatter_add_groups(exb, acc, dstb2, ng, s1)
                return 0
            lax.fori_loop(0, nch, chunk, 0)
            plsc.subcore_barrier()
            _stripe_drain(acc, dp_h.at[cid].at[pl.ds(lo, n2)], sid, s0, tail)
            plsc.subcore_barrier()

    return k(src, dst, asp, adp, cvec, zeros16)


def _sc_pass_b(src, dst, exT, h4, zeros32):
    e = src.shape[0]
    n = h4.shape[0] // HEADS
    n2 = n // 2
    ew = e // NW
    c = 400
    ng = c // GRP
    nch = ew // c
    s0 = (n2 // NS) & ~7
    tail = n2 - NS * s0

    @functools.partial(
        pl.kernel,
        out_type=jax.ShapeDtypeStruct((HEADS, NC, n, HID), jnp.float32),
        mesh=_mesh(),
        compiler_params=pltpu.CompilerParams(use_tc_tiling_on_sc=False),
        scratch_types=[
            pltpu.VMEM((c,), jnp.int32),
            pltpu.VMEM((ng, GRP), jnp.int32),
            pltpu.VMEM((ng, GRP), jnp.int32),
            pltpu.VMEM((c,), jnp.int32),
            pltpu.VMEM((HEADS * c,), jnp.float32),
            pltpu.VMEM((c, HID), jnp.float32),
            pltpu.VMEM_SHARED((n2 + 8, HID), jnp.float32),
            pltpu.SemaphoreType.DMA,
            pltpu.SemaphoreType.DMA,
        ],
    )
    def k(src_h, dst_h, exT_h, h4_h, z_h, out_h,
          srcb, dstb, dstb2, idxb, exw, rows, acc, s1, s2):
        cid = lax.axis_index("c")
        sid = lax.axis_index("s")
        wid = sid * NC + cid
        for hd in range(HEADS):
            def sweep(r, _):
                lo = r * n2
                _stripe_zero(z_h, acc, sid, s0, tail + 8)
                plsc.subcore_barrier()

                def chunk(i, _):
                    base = wid * ew + i * c
                    pltpu.sync_copy(src_h.at[pl.ds(base, c)], srcb)
                    _load_dst_groups(dst_h, dstb, base, ng, s2)
                    pltpu.sync_copy(
                        exT_h.at[pl.ds(base * HEADS, c * HEADS)], exw)

                    @plsc.parallel_loop(0, c, L, unroll=4)
                    def _(j0):
                        v = srcb[pl.ds(j0, L)]
                        idxb[pl.ds(j0, L)] = v * HEADS + hd
                    _fire_gathers(h4_h, idxb, rows, ng, s1)
                    _range_idx(dstb, dstb2, lo, n2, ng)
                    _drain_gathers(h4_h, idxb, rows, ng, s1)

                    @plsc.parallel_loop(0, c, 4, unroll=4)
                    def _(j0):
                        ex16 = exw[pl.ds(j0 * HEADS, L)]
                        for t in range(4):
                            s = ex16[t * HEADS + hd]
                            rows[j0 + t, pl.ds(0, L)] = (
                                rows[j0 + t, pl.ds(0, L)] * s)
                            rows[j0 + t, pl.ds(L, L)] = (
                                rows[j0 + t, pl.ds(L, L)] * s)
                    _scatter_add_groups(rows, acc, dstb2, ng, s1)
                    return 0
                lax.fori_loop(0, nch, chunk, 0)
                plsc.subcore_barrier()
                _stripe_drain(acc, out_h.at[hd, cid].at[pl.ds(lo, n2)],
                              sid, s0, tail)
                plsc.subcore_barrier()
                return 0
            lax.fori_loop(0, 2, sweep, 0)

    return k(src, dst, exT, h4, zeros32)


def _sc_pass_d(src, dst, hfin):
    e = src.shape[0]
    ew = e // NW
    c = 400
    ng = c // GRP
    nch = ew // c

    @functools.partial(
        pl.kernel,
        out_type=[
            jax.ShapeDtypeStruct((e, HID), jnp.float32),
            jax.ShapeDtypeStruct((e, HID), jnp.float32),
        ],
        mesh=_mesh(),
        compiler_params=pltpu.CompilerParams(
            use_tc_tiling_on_sc=False, internal_scratch_in_bytes=1 << 16),
        scratch_types=[
            pltpu.VMEM((c,), jnp.int32),
            pltpu.VMEM((c,), jnp.int32),
            pltpu.VMEM((c, HID), jnp.float32),
            pltpu.VMEM((c, HID), jnp.float32),
            pltpu.SemaphoreType.DMA,
            pltpu.SemaphoreType.DMA,
        ],
    )
    def k(src_h, dst_h, hf_h, hs_h, hd_h, srcb, dstb, rs, rd, s1, s2):
        cid = lax.axis_index("c")
        sid = lax.axis_index("s")
        wid = sid * NC + cid

        def chunk(i, _):
            base = wid * ew + i * c
            pltpu.sync_copy(src_h.at[pl.ds(base, c)], srcb)
            pltpu.sync_copy(dst_h.at[pl.ds(base, c)], dstb)
            cps = []
            for g in range(ng):
                cps.append(pltpu.async_copy(
                    hf_h.at[srcb.at[pl.ds(g * GRP, GRP)]],
                    rs.at[pl.ds(g * GRP, GRP)], s1))
                cps.append(pltpu.async_copy(
                    hf_h.at[dstb.at[pl.ds(g * GRP, GRP)]],
                    rd.at[pl.ds(g * GRP, GRP)], s2))
            for cp in cps:
                cp.wait()
            pltpu.sync_copy(rs, hs_h.at[pl.ds(base, c)])
            pltpu.sync_copy(rd, hd_h.at[pl.ds(base, c)])
            return 0
        lax.fori_loop(0, nch, chunk, 0)

    return k(src, dst, hfin)


def _elu(x):
    return jnp.where(x > 0, x, jnp.exp(jnp.minimum(x, 0.0)) - 1.0)


def _softmax_pieces(dp_ref, asp_ref, adp_ref, c_ref, h_ref, mp_ref):
    sa = asp_ref[:, :HEADS] + adp_ref[:, :HEADS]
    ex_ii = jnp.exp(jnp.maximum(sa, 0.2 * sa) - c_ref[...])
    denom = dp_ref[0][:, :HEADS] + dp_ref[1][:, :HEADS] + ex_ii
    rden = 1.0 / (denom + 1e-16)
    pieces = []
    for hd in range(HEADS):
        aggh = mp_ref[hd, 0] + mp_ref[hd, 1]
        aggh = aggh + h_ref[:, HID * hd:HID * (hd + 1)] * ex_ii[:, hd:hd + 1]
        pieces.append(aggh * rden[:, hd:hd + 1])
    return pieces


def _combine1_body(dp_ref, asp_ref, adp_ref, c_ref, h_ref, mp_ref,
                   b_ref, w2_ref, as2_ref, ad2_ref,
                   h2_ref, a2sp_ref, a2dp_ref, ms_ref, md_ref):
    i = pl.program_id(0)
    pieces = _softmax_pieces(dp_ref, asp_ref, adp_ref, c_ref, h_ref, mp_ref)
    o = _elu(jnp.concatenate(pieces, axis=1) + b_ref[...])
    h2 = jnp.dot(o, w2_ref[...], preferred_element_type=jnp.float32)
    h2_ref[...] = h2
    a_s = jnp.dot(h2, as2_ref[...], preferred_element_type=jnp.float32)
    a_d = jnp.dot(h2, ad2_ref[...], preferred_element_type=jnp.float32)
    bn = a_s.shape[0]
    pad = jnp.zeros((bn, L - HEADS), jnp.float32)
    a2sp_ref[...] = jnp.concatenate([a_s, pad], axis=1)
    a2dp_ref[...] = jnp.concatenate([a_d, pad], axis=1)
    bs = jnp.max(a_s, axis=0, keepdims=True)
    bd = jnp.max(a_d, axis=0, keepdims=True)

    @pl.when(i == 0)
    def _():
        ms_ref[...] = bs
        md_ref[...] = bd

    @pl.when(i != 0)
    def _():
        ms_ref[...] = jnp.maximum(ms_ref[...], bs)
        md_ref[...] = jnp.maximum(md_ref[...], bd)


def _combine1(dp, asp, adp, c, H, mp, b, W2, As2, Ad2, bn=1000):
    n = H.shape[0]
    return pl.pallas_call(
        _combine1_body,
        grid=(n // bn,),
        in_specs=[
            pl.BlockSpec((NC, bn, L), lambda i: (0, i, 0)),
            pl.BlockSpec((bn, L), lambda i: (i, 0)),
            pl.BlockSpec((bn, L), lambda i: (i, 0)),
            pl.BlockSpec((1, HEADS), lambda i: (0, 0)),
            pl.BlockSpec((bn, F), lambda i: (i, 0)),
            pl.BlockSpec((HEADS, NC, bn, HID), lambda i: (0, 0, i, 0)),
            pl.BlockSpec((1, F), lambda i: (0, 0)),
            pl.BlockSpec((F, F), lambda i: (0, 0)),
            pl.BlockSpec((F, HEADS), lambda i: (0, 0)),
            pl.BlockSpec((F, HEADS), lambda i: (0, 0)),
        ],
        out_specs=[
            pl.BlockSpec((bn, F), lambda i: (i, 0)),
            pl.BlockSpec((bn, L), lambda i: (i, 0)),
            pl.BlockSpec((bn, L), lambda i: (i, 0)),
            pl.BlockSpec((1, HEADS), lambda i: (0, 0)),
            pl.BlockSpec((1, HEADS), lambda i: (0, 0)),
        ],
        out_shape=[
            jax.ShapeDtypeStruct((n, F), jnp.float32),
            jax.ShapeDtypeStruct((n, L), jnp.float32),
            jax.ShapeDtypeStruct((n, L), jnp.float32),
            jax.ShapeDtypeStruct((1, HEADS), jnp.float32),
            jax.ShapeDtypeStruct((1, HEADS), jnp.float32),
        ],
    )(dp, asp, adp, c, H, mp, b, W2, As2, Ad2)


def _combine2_body(dp_ref, asp_ref, adp_ref, c_ref, h_ref, mp_ref, b_ref,
                   out_ref):
    pieces = _softmax_pieces(dp_ref, asp_ref, adp_ref, c_ref, h_ref, mp_ref)
    acc = pieces[0]
    for p in pieces[1:]:
        acc = acc + p
    out_ref[...] = _elu(acc * (1.0 / HEADS) + b_ref[...])


def _combine2(dp, asp, adp, c, H, mp, b, bn=1000):
    n = H.shape[0]
    return pl.pallas_call(
        _combine2_body,
        grid=(n // bn,),
        in_specs=[
            pl.BlockSpec((NC, bn, L), lambda i: (0, i, 0)),
            pl.BlockSpec((bn, L), lambda i: (i, 0)),
            pl.BlockSpec((bn, L), lambda i: (i, 0)),
            pl.BlockSpec((1, HEADS), lambda i: (0, 0)),
            pl.BlockSpec((bn, F), lambda i: (i, 0)),
            pl.BlockSpec((HEADS, NC, bn, HID), lambda i: (0, 0, i, 0)),
            pl.BlockSpec((1, HID), lambda i: (0, 0)),
        ],
        out_specs=pl.BlockSpec((bn, HID), lambda i: (i, 0)),
        out_shape=jax.ShapeDtypeStruct((n, HID), jnp.float32),
    )(dp, asp, adp, c, H, mp, b)


def _decoder_body(hs_ref, hd_ref, ea_ref, w1s_ref, w1d_ref, w1e_ref, b1_ref,
                  w2_ref, b2_ref, w3_ref, b3_ref, out_ref):
    z = jnp.dot(hs_ref[...], w1s_ref[...], preferred_element_type=jnp.float32)
    z += jnp.dot(hd_ref[...], w1d_ref[...], preferred_element_type=jnp.float32)
    z += jnp.dot(ea_ref[...], w1e_ref[...], preferred_element_type=jnp.float32)
    z = jnp.maximum(z + b1_ref[...], 0.0)
    z = jnp.maximum(
        jnp.dot(z, w2_ref[...], preferred_element_type=jnp.float32)
        + b2_ref[...], 0.0)
    out_ref[...] = (
        jnp.dot(z, w3_ref[...], preferred_element_type=jnp.float32)
        + b3_ref[...])


def _decoder(hs, hd, ea, Wd1, bd1, Wd2, bd2, Wd3, bd3, be=8000):
    e = hs.shape[0]
    w1s, w1d, w1e = Wd1[:HID], Wd1[HID:2 * HID], Wd1[2 * HID:]
    return pl.pallas_call(
        _decoder_body,
        grid=(e // be,),
        in_specs=[
            pl.BlockSpec((be, HID), lambda i: (i, 0)),
            pl.BlockSpec((be, HID), lambda i: (i, 0)),
            pl.BlockSpec((be, 5), lambda i: (i, 0)),
            pl.BlockSpec((HID, 2 * HID), lambda i: (0, 0)),
            pl.BlockSpec((HID, 2 * HID), lambda i: (0, 0)),
            pl.BlockSpec((5, 2 * HID), lambda i: (0, 0)),
            pl.BlockSpec((1, 2 * HID), lambda i: (0, 0)),
            pl.BlockSpec((2 * HID, HID), lambda i: (0, 0)),
            pl.BlockSpec((1, HID), lambda i: (0, 0)),
            pl.BlockSpec((HID, 4), lambda i: (0, 0)),
            pl.BlockSpec((1, 4), lambda i: (0, 0)),
        ],
        out_specs=pl.BlockSpec((be, 4), lambda i: (i, 0)),
        out_shape=jax.ShapeDtypeStruct((e, 4), jnp.float32),
    )(hs, hd, ea, w1s, w1d, w1e, bd1[None], Wd2, bd2[None], Wd3, bd3[None])


def _block_diag_att(att):
    m = jnp.zeros((HEADS, HID, HEADS), dtype=att.dtype)
    m = m.at[jnp.arange(HEADS), :, jnp.arange(HEADS)].set(att[0])
    return m.reshape(F, HEADS)


def _shift(ms, md):
    sa = ms + md
    c = jnp.maximum(sa, 0.2 * sa)
    cvec = jnp.tile(c[0], HEADS)
    return c, cvec


def kernel(x, edge_index, edge_attr, u, W1, att1_src, att1_dst, b1,
           W2, att2_src, att2_dst, b2, Wd1, bd1, Wd2, bd2, Wd3, bd3):
    n = x.shape[0]
    e = edge_index.shape[1]
    src = edge_index[0]
    dst = edge_index[1]
    del e
    zeros16 = jnp.zeros((n, L), jnp.float32)
    zeros32 = jnp.zeros((n, HID), jnp.float32)

    u_node = jnp.broadcast_to(u, (n, u.shape[1]))
    h0 = jnp.concatenate([x, u_node], axis=-1)

    H1, a1sp, a1dp, ms1, md1 = _proj(
        h0, W1, _block_diag_att(att1_src), _block_diag_att(att1_dst))
    c1, c1vec = _shift(ms1, md1)
    exT1, dp1 = _sc_pass_a(src, dst, a1sp, a1dp, c1vec, zeros16)
    mp1 = _sc_pass_b(src, dst, exT1, H1.reshape(HEADS * n, HID), zeros32)
    H2, a2sp, a2dp, ms2, md2 = _combine1(
        dp1, a1sp, a1dp, c1, H1, mp1, b1[None], W2,
        _block_diag_att(att2_src), _block_diag_att(att2_dst))

    c2, c2vec = _shift(ms2, md2)
    exT2, dp2 = _sc_pass_a(src, dst, a2sp, a2dp, c2vec, zeros16)
    mp2 = _sc_pass_b(src, dst, exT2, H2.reshape(HEADS * n, HID), zeros32)
    hfin = _combine2(dp2, a2sp, a2dp, c2, H2, mp2, b2[None])

    hs, hd = _sc_pass_d(src, dst, hfin)
    return _decoder(hs, hd, edge_attr, Wd1, bd1, Wd2, bd2, Wd3, bd3)


# speedup vs baseline: 38.0225x; 1.0291x over previous
import functools

import jax
import jax.numpy as jnp
from jax import lax
from jax.experimental import pallas as pl
from jax.experimental.pallas import tpu as pltpu
from jax.experimental.pallas import tpu_sc as plsc

HEADS = 4
HID = 32
F = HEADS * HID
L = 16
NC = 2
NS = 16
NW = NC * NS
GRP = 80


def _mesh():
    return plsc.VectorSubcoreMesh(
        core_axis_name="c", subcore_axis_name="s",
        num_cores=NC, num_subcores=NS)


def _proj_body(h0_ref, w_ref, as_ref, ad_ref,
               h_ref, asp_ref, adp_ref, ms_ref, md_ref):
    i = pl.program_id(0)
    h = jnp.dot(h0_ref[...], w_ref[...], preferred_element_type=jnp.float32)
    h_ref[...] = h
    a_s = jnp.dot(h, as_ref[...], preferred_element_type=jnp.float32)
    a_d = jnp.dot(h, ad_ref[...], preferred_element_type=jnp.float32)
    bn = a_s.shape[0]
    pad = jnp.zeros((bn, L - HEADS), jnp.float32)
    asp_ref[...] = jnp.concatenate([a_s, pad], axis=1)
    adp_ref[...] = jnp.concatenate([a_d, pad], axis=1)
    bs = jnp.max(a_s, axis=0, keepdims=True)
    bd = jnp.max(a_d, axis=0, keepdims=True)

    @pl.when(i == 0)
    def _():
        ms_ref[...] = bs
        md_ref[...] = bd

    @pl.when(i != 0)
    def _():
        ms_ref[...] = jnp.maximum(ms_ref[...], bs)
        md_ref[...] = jnp.maximum(md_ref[...], bd)


def _proj(h0, W, As, Ad, bn=1000):
    n, k = h0.shape
    return pl.pallas_call(
        _proj_body,
        grid=(n // bn,),
        in_specs=[
            pl.BlockSpec((bn, k), lambda i: (i, 0)),
            pl.BlockSpec((k, F), lambda i: (0, 0)),
            pl.BlockSpec((F, HEADS), lambda i: (0, 0)),
            pl.BlockSpec((F, HEADS), lambda i: (0, 0)),
        ],
        out_specs=[
            pl.BlockSpec((bn, F), lambda i: (i, 0)),
            pl.BlockSpec((bn, L), lambda i: (i, 0)),
            pl.BlockSpec((bn, L), lambda i: (i, 0)),
            pl.BlockSpec((1, HEADS), lambda i: (0, 0)),
            pl.BlockSpec((1, HEADS), lambda i: (0, 0)),
        ],
        out_shape=[
            jax.ShapeDtypeStruct((n, F), jnp.float32),
            jax.ShapeDtypeStruct((n, L), jnp.float32),
            jax.ShapeDtypeStruct((n, L), jnp.float32),
            jax.ShapeDtypeStruct((1, HEADS), jnp.float32),
            jax.ShapeDtypeStruct((1, HEADS), jnp.float32),
        ],
    )(h0, W, As, Ad)


def _stripe_zero(z_h, acc, sid, s0, tail):
    pltpu.sync_copy(z_h.at[pl.ds(sid * s0, s0)], acc.at[pl.ds(sid * s0, s0)])

    @pl.when(sid == NS - 1)
    def _():
        pltpu.sync_copy(z_h.at[pl.ds(NS * s0, tail)],
                        acc.at[pl.ds(NS * s0, tail)])


def _stripe_drain(acc, dst_ref, sid, s0, tail):
    pltpu.sync_copy(acc.at[pl.ds(sid * s0, s0)],
                    dst_ref.at[pl.ds(sid * s0, s0)])

    @pl.when(sid == NS - 1)
    def _():
        pltpu.sync_copy(acc.at[pl.ds(NS * s0, tail)],
                        dst_ref.at[pl.ds(NS * s0, tail)])


def _load_dst_groups(dst_h, dstb, base, ng, sem):
    def fire(g, _):
        pltpu.async_copy(dst_h.at[pl.ds(base + g * GRP, GRP)],
                         dstb.at[g], sem)
        return 0
    lax.fori_loop(0, ng, fire, 0)

    def drain(g, _):
        pltpu.make_async_copy(dst_h.at[pl.ds(base, GRP)],
                              dstb.at[0], sem).wait()
        return 0
    lax.fori_loop(0, ng, drain, 0)


def _range_idx(dstb, dstb2, lo, n2, ng):
    kk = GRP // L

    @plsc.parallel_loop(0, ng * kk, 1, unroll=4)
    def _(q):
        g = q // kk
        k = q % kk
        dv = dstb[g, pl.ds(k * L, L)]
        ok = (dv >= lo) & (dv < lo + n2)
        dstb2[g, pl.ds(k * L, L)] = jnp.where(ok, dv - lo, n2)


def _fire_gathers(table_h, idx_ref, rows_ref, ng, sem):
    def fire(g, _):
        pltpu.async_copy(table_h.at[idx_ref.at[pl.ds(g * GRP, GRP)]],
                         rows_ref.at[pl.ds(g * GRP, GRP)], sem)
        return 0
    lax.fori_loop(0, ng, fire, 0)


def _drain_gathers(table_h, idx_ref, rows_ref, ng, sem):
    def drain(g, _):
        pltpu.make_async_copy(table_h.at[idx_ref.at[pl.ds(0, GRP)]],
                              rows_ref.at[pl.ds(0, GRP)], sem).wait()
        return 0
    lax.fori_loop(0, ng, drain, 0)


def _scatter_add_groups(rows_ref, acc, idx2_ref, ng, sem):
    def fire(g, _):
        pltpu.async_copy(rows_ref.at[pl.ds(g * GRP, GRP)],
                         acc.at[idx2_ref.at[g]], sem, add=True)
        return 0
    lax.fori_loop(0, ng, fire, 0)

    def drain(g, _):
        pltpu.make_async_copy(rows_ref.at[pl.ds(0, GRP)],
                              acc.at[idx2_ref.at[0]], sem).wait()
        return 0
    lax.fori_loop(0, ng, drain, 0)


def _sc_pass_a(src, dst, asp, adp, cvec, zeros16):
    e = src.shape[0]
    n = asp.shape[0]
    n2 = n // 2
    ew = e // NW
    c = 400
    ng = c // GRP
    nch = ew // c
    s0 = (n2 // NS) & ~7
    tail = n2 - NS * s0

    @functools.partial(
        pl.kernel,
        out_type=[
            jax.ShapeDtypeStruct((HEADS * e,), jnp.float32),
            jax.ShapeDtypeStruct((NC, n, L), jnp.float32),
        ],
        mesh=_mesh(),
        compiler_params=pltpu.CompilerParams(use_tc_tiling_on_sc=False),
        scratch_types=[
            pltpu.VMEM((c,), jnp.int32),
            pltpu.VMEM((ng, GRP), jnp.int32),
            pltpu.VMEM((ng, GRP), jnp.int32),
            pltpu.VMEM((c, L), jnp.float32),
            pltpu.VMEM((c, L), jnp.float32),
            pltpu.VMEM((c, L), jnp.float32),
            pltpu.VMEM((HEADS * c,), jnp.float32),
            pltpu.VMEM((L,), jnp.float32),
            pltpu.VMEM_SHARED((n2 + 8, L), jnp.float32),
            pltpu.SemaphoreType.DMA,
            pltpu.SemaphoreType.DMA,
        ],
    )
    def k(src_h, dst_h, asp_h, adp_h, cvec_h, z_h, exT_h, dp_h,
          srcb, dstb, dstb2, asb, adb, exb, exTb, cb, acc, s1, s2):
        cid = lax.axis_index("c")
        sid = lax.axis_index("s")
        wid = sid * NC + cid
        pltpu.sync_copy(cvec_h, cb)
        cv = cb[...]
        iota = lax.iota(jnp.int32, L)
        iotam4 = iota % HEADS
        m4 = iota < 4
        m8 = iota < 8
        m12 = iota < 12

        for r in range(2):
            lo = r * n2
            _stripe_zero(z_h, acc, sid, s0, tail + 8)
            plsc.subcore_barrier()

            def chunk(i, _):
                base = wid * ew + i * c
                pltpu.sync_copy(src_h.at[pl.ds(base, c)], srcb)
                _load_dst_groups(dst_h, dstb, base, ng, s2)
                _fire_gathers(asp_h, srcb, asb, ng, s1)

                def fire_ad(g, _):
                    pltpu.async_copy(adp_h.at[dstb.at[g]],
                                     adb.at[pl.ds(g * GRP, GRP)], s2)
                    return 0
                lax.fori_loop(0, ng, fire_ad, 0)
                _range_idx(dstb, dstb2, lo, n2, ng)
                _drain_gathers(asp_h, srcb, asb, ng, s1)

                def drain_ad(g, _):
                    pltpu.make_async_copy(adp_h.at[dstb.at[0]],
                                          adb.at[pl.ds(0, GRP)], s2).wait()
                    return 0
                lax.fori_loop(0, ng, drain_ad, 0)

                @plsc.parallel_loop(0, c, 4, unroll=4)
                def _(j0):
                    gs = []
                    for t in range(4):
                        j = j0 + t
                        s = asb[j] + adb[j]
                        v = jnp.exp(jnp.maximum(s, 0.2 * s) - cv)
                        exb[j] = v
                        if r == 0:
                            gs.append(
                                v.at[iotam4].get(mode="promise_in_bounds"))
                    if r == 0:
                        q = jnp.where(m4, gs[0],
                                      jnp.where(m8, gs[1],
                                                jnp.where(m12, gs[2], gs[3])))
                        exTb[pl.ds(j0 * HEADS, L)] = q

                if r == 0:
                    pltpu.sync_copy(exTb,
                                    exT_h.at[pl.ds(base * HEADS, c * HEADS)])
                _scatter_add_groups(exb, acc, dstb2, ng, s1)
                return 0
            lax.fori_loop(0, nch, chunk, 0)
            plsc.subcore_barrier()
            _stripe_drain(acc, dp_h.at[cid].at[pl.ds(lo, n2)], sid, s0, tail)
            plsc.subcore_barrier()

    return k(src, dst, asp, adp, cvec, zeros16)


def _sc_pass_b(srcx, dst, exT, h4, zeros32):
    e = srcx.shape[0] // HEADS
    n = h4.shape[0] // HEADS
    n2 = n // 2
    ew = e // NW
    c = 400
    ng = c // GRP
    nch = ew // c
    s0 = (n2 // NS) & ~7
    tail = n2 - NS * s0

    @functools.partial(
        pl.kernel,
        out_type=jax.ShapeDtypeStruct((HEADS, NC, n, HID), jnp.float32),
        mesh=_mesh(),
        compiler_params=pltpu.CompilerParams(use_tc_tiling_on_sc=False),
        scratch_types=[
            pltpu.VMEM((ng, GRP), jnp.int32),
            pltpu.VMEM((ng, GRP), jnp.int32),
            pltpu.VMEM((c,), jnp.int32),
            pltpu.VMEM((HEADS * c,), jnp.float32),
            pltpu.VMEM((c, HID), jnp.float32),
            pltpu.VMEM_SHARED((n2 + 8, HID), jnp.float32),
            pltpu.SemaphoreType.DMA,
            pltpu.SemaphoreType.DMA,
        ],
    )
    def k(srcx_h, dst_h, exT_h, h4_h, z_h, out_h,
          dstb, dstb2, idxb, exw, rows, acc, s1, s2):
        cid = lax.axis_index("c")
        sid = lax.axis_index("s")
        wid = sid * NC + cid
        for hd in range(HEADS):
            def sweep(r, _):
                lo = r * n2
                _stripe_zero(z_h, acc, sid, s0, tail + 8)
                plsc.subcore_barrier()

                def chunk(i, _):
                    base = wid * ew + i * c
                    pltpu.sync_copy(srcx_h.at[pl.ds(hd * e + base, c)], idxb)
                    _load_dst_groups(dst_h, dstb, base, ng, s2)
                    exc = pltpu.async_copy(
                        exT_h.at[pl.ds(base * HEADS, c * HEADS)], exw, s2)
                    _fire_gathers(h4_h, idxb, rows, ng, s1)
                    _range_idx(dstb, dstb2, lo, n2, ng)
                    exc.wait()
                    _drain_gathers(h4_h, idxb, rows, ng, s1)

                    @plsc.parallel_loop(0, c, 4, unroll=4)
                    def _(j0):
                        ex16 = exw[pl.ds(j0 * HEADS, L)]
                        for t in range(4):
                            s = ex16[t * HEADS + hd]
                            rows[j0 + t, pl.ds(0, L)] = (
                                rows[j0 + t, pl.ds(0, L)] * s)
                            rows[j0 + t, pl.ds(L, L)] = (
                                rows[j0 + t, pl.ds(L, L)] * s)
                    _scatter_add_groups(rows, acc, dstb2, ng, s1)
                    return 0
                lax.fori_loop(0, nch, chunk, 0)
                plsc.subcore_barrier()
                _stripe_drain(acc, out_h.at[hd, cid].at[pl.ds(lo, n2)],
                              sid, s0, tail)
                plsc.subcore_barrier()
                return 0
            lax.fori_loop(0, 2, sweep, 0)

    return k(srcx, dst, exT, h4, zeros32)


def _sc_pass_d(src, dst, hfin):
    e = src.shape[0]
    ew = e // NW
    c = 400
    ng = c // GRP
    nch = ew // c

    @functools.partial(
        pl.kernel,
        out_type=[
            jax.ShapeDtypeStruct((e, HID), jnp.float32),
            jax.ShapeDtypeStruct((e, HID), jnp.float32),
        ],
        mesh=_mesh(),
        compiler_params=pltpu.CompilerParams(
            use_tc_tiling_on_sc=False, internal_scratch_in_bytes=1 << 16),
        scratch_types=[
            pltpu.VMEM((c,), jnp.int32),
            pltpu.VMEM((c,), jnp.int32),
            pltpu.VMEM((c, HID), jnp.float32),
            pltpu.VMEM((c, HID), jnp.float32),
            pltpu.SemaphoreType.DMA,
            pltpu.SemaphoreType.DMA,
        ],
    )
    def k(src_h, dst_h, hf_h, hs_h, hd_h, srcb, dstb, rs, rd, s1, s2):
        cid = lax.axis_index("c")
        sid = lax.axis_index("s")
        wid = sid * NC + cid

        def chunk(i, _):
            base = wid * ew + i * c
            pltpu.sync_copy(src_h.at[pl.ds(base, c)], srcb)
            pltpu.sync_copy(dst_h.at[pl.ds(base, c)], dstb)
            cps = []
            for g in range(ng):
                cps.append(pltpu.async_copy(
                    hf_h.at[srcb.at[pl.ds(g * GRP, GRP)]],
                    rs.at[pl.ds(g * GRP, GRP)], s1))
                cps.append(pltpu.async_copy(
                    hf_h.at[dstb.at[pl.ds(g * GRP, GRP)]],
                    rd.at[pl.ds(g * GRP, GRP)], s2))
            for cp in cps:
                cp.wait()
            pltpu.sync_copy(rs, hs_h.at[pl.ds(base, c)])
            pltpu.sync_copy(rd, hd_h.at[pl.ds(base, c)])
            return 0
        lax.fori_loop(0, nch, chunk, 0)

    return k(src, dst, hfin)


def _elu(x):
    return jnp.where(x > 0, x, jnp.exp(jnp.minimum(x, 0.0)) - 1.0)


def _softmax_pieces(dp_ref, asp_ref, adp_ref, c_ref, h_ref, mp_ref):
    sa = asp_ref[:, :HEADS] + adp_ref[:, :HEADS]
    ex_ii = jnp.exp(jnp.maximum(sa, 0.2 * sa) - c_ref[...])
    denom = dp_ref[0][:, :HEADS] + dp_ref[1][:, :HEADS] + ex_ii
    rden = 1.0 / (denom + 1e-16)
    pieces = []
    for hd in range(HEADS):
        aggh = mp_ref[hd, 0] + mp_ref[hd, 1]
        aggh = aggh + h_ref[:, HID * hd:HID * (hd + 1)] * ex_ii[:, hd:hd + 1]
        pieces.append(aggh * rden[:, hd:hd + 1])
    return pieces


def _combine1_body(dp_ref, asp_ref, adp_ref, c_ref, h_ref, mp_ref,
                   b_ref, w2_ref, as2_ref, ad2_ref,
                   h2_ref, a2sp_ref, a2dp_ref, ms_ref, md_ref):
    i = pl.program_id(0)
    pieces = _softmax_pieces(dp_ref, asp_ref, adp_ref, c_ref, h_ref, mp_ref)
    o = _elu(jnp.concatenate(pieces, axis=1) + b_ref[...])
    h2 = jnp.dot(o, w2_ref[...], preferred_element_type=jnp.float32)
    h2_ref[...] = h2
    a_s = jnp.dot(h2, as2_ref[...], preferred_element_type=jnp.float32)
    a_d = jnp.dot(h2, ad2_ref[...], preferred_element_type=jnp.float32)
    bn = a_s.shape[0]
    pad = jnp.zeros((bn, L - HEADS), jnp.float32)
    a2sp_ref[...] = jnp.concatenate([a_s, pad], axis=1)
    a2dp_ref[...] = jnp.concatenate([a_d, pad], axis=1)
    bs = jnp.max(a_s, axis=0, keepdims=True)
    bd = jnp.max(a_d, axis=0, keepdims=True)

    @pl.when(i == 0)
    def _():
        ms_ref[...] = bs
        md_ref[...] = bd

    @pl.when(i != 0)
    def _():
        ms_ref[...] = jnp.maximum(ms_ref[...], bs)
        md_ref[...] = jnp.maximum(md_ref[...], bd)


def _combine1(dp, asp, adp, c, H, mp, b, W2, As2, Ad2, bn=1000):
    n = H.shape[0]
    return pl.pallas_call(
        _combine1_body,
        grid=(n // bn,),
        in_specs=[
            pl.BlockSpec((NC, bn, L), lambda i: (0, i, 0)),
            pl.BlockSpec((bn, L), lambda i: (i, 0)),
            pl.BlockSpec((bn, L), lambda i: (i, 0)),
            pl.BlockSpec((1, HEADS), lambda i: (0, 0)),
            pl.BlockSpec((bn, F), lambda i: (i, 0)),
            pl.BlockSpec((HEADS, NC, bn, HID), lambda i: (0, 0, i, 0)),
            pl.BlockSpec((1, F), lambda i: (0, 0)),
            pl.BlockSpec((F, F), lambda i: (0, 0)),
            pl.BlockSpec((F, HEADS), lambda i: (0, 0)),
            pl.BlockSpec((F, HEADS), lambda i: (0, 0)),
        ],
        out_specs=[
            pl.BlockSpec((bn, F), lambda i: (i, 0)),
            pl.BlockSpec((bn, L), lambda i: (i, 0)),
            pl.BlockSpec((bn, L), lambda i: (i, 0)),
            pl.BlockSpec((1, HEADS), lambda i: (0, 0)),
            pl.BlockSpec((1, HEADS), lambda i: (0, 0)),
        ],
        out_shape=[
            jax.ShapeDtypeStruct((n, F), jnp.float32),
            jax.ShapeDtypeStruct((n, L), jnp.float32),
            jax.ShapeDtypeStruct((n, L), jnp.float32),
            jax.ShapeDtypeStruct((1, HEADS), jnp.float32),
            jax.ShapeDtypeStruct((1, HEADS), jnp.float32),
        ],
    )(dp, asp, adp, c, H, mp, b, W2, As2, Ad2)


def _combine2_body(dp_ref, asp_ref, adp_ref, c_ref, h_ref, mp_ref, b_ref,
                   out_ref):
    pieces = _softmax_pieces(dp_ref, asp_ref, adp_ref, c_ref, h_ref, mp_ref)
    acc = pieces[0]
    for p in pieces[1:]:
        acc = acc + p
    out_ref[...] = _elu(acc * (1.0 / HEADS) + b_ref[...])


def _combine2(dp, asp, adp, c, H, mp, b, bn=1000):
    n = H.shape[0]
    return pl.pallas_call(
        _combine2_body,
        grid=(n // bn,),
        in_specs=[
            pl.BlockSpec((NC, bn, L), lambda i: (0, i, 0)),
            pl.BlockSpec((bn, L), lambda i: (i, 0)),
            pl.BlockSpec((bn, L), lambda i: (i, 0)),
            pl.BlockSpec((1, HEADS), lambda i: (0, 0)),
            pl.BlockSpec((bn, F), lambda i: (i, 0)),
            pl.BlockSpec((HEADS, NC, bn, HID), lambda i: (0, 0, i, 0)),
            pl.BlockSpec((1, HID), lambda i: (0, 0)),
        ],
        out_specs=pl.BlockSpec((bn, HID), lambda i: (i, 0)),
        out_shape=jax.ShapeDtypeStruct((n, HID), jnp.float32),
    )(dp, asp, adp, c, H, mp, b)


def _decoder_body(hs_ref, hd_ref, ea_ref, w1s_ref, w1d_ref, w1e_ref, b1_ref,
                  w2_ref, b2_ref, w3_ref, b3_ref, out_ref):
    z = jnp.dot(hs_ref[...], w1s_ref[...], preferred_element_type=jnp.float32)
    z += jnp.dot(hd_ref[...], w1d_ref[...], preferred_element_type=jnp.float32)
    z += jnp.dot(ea_ref[...], w1e_ref[...], preferred_element_type=jnp.float32)
    z = jnp.maximum(z + b1_ref[...], 0.0)
    z = jnp.maximum(
        jnp.dot(z, w2_ref[...], preferred_element_type=jnp.float32)
        + b2_ref[...], 0.0)
    out_ref[...] = (
        jnp.dot(z, w3_ref[...], preferred_element_type=jnp.float32)
        + b3_ref[...])


def _decoder(hs, hd, ea, Wd1, bd1, Wd2, bd2, Wd3, bd3, be=8000):
    e = hs.shape[0]
    w1s, w1d, w1e = Wd1[:HID], Wd1[HID:2 * HID], Wd1[2 * HID:]
    return pl.pallas_call(
        _decoder_body,
        grid=(e // be,),
        in_specs=[
            pl.BlockSpec((be, HID), lambda i: (i, 0)),
            pl.BlockSpec((be, HID), lambda i: (i, 0)),
            pl.BlockSpec((be, 5), lambda i: (i, 0)),
            pl.BlockSpec((HID, 2 * HID), lambda i: (0, 0)),
            pl.BlockSpec((HID, 2 * HID), lambda i: (0, 0)),
            pl.BlockSpec((5, 2 * HID), lambda i: (0, 0)),
            pl.BlockSpec((1, 2 * HID), lambda i: (0, 0)),
            pl.BlockSpec((2 * HID, HID), lambda i: (0, 0)),
            pl.BlockSpec((1, HID), lambda i: (0, 0)),
            pl.BlockSpec((HID, 4), lambda i: (0, 0)),
            pl.BlockSpec((1, 4), lambda i: (0, 0)),
        ],
        out_specs=pl.BlockSpec((be, 4), lambda i: (i, 0)),
        out_shape=jax.ShapeDtypeStruct((e, 4), jnp.float32),
    )(hs, hd, ea, w1s, w1d, w1e, bd1[None], Wd2, bd2[None], Wd3, bd3[None])


def _block_diag_att(att):
    m = jnp.zeros((HEADS, HID, HEADS), dtype=att.dtype)
    m = m.at[jnp.arange(HEADS), :, jnp.arange(HEADS)].set(att[0])
    return m.reshape(F, HEADS)


def _shift(ms, md):
    sa = ms + md
    c = jnp.maximum(sa, 0.2 * sa)
    cvec = jnp.tile(c[0], HEADS)
    return c, cvec


def kernel(x, edge_index, edge_attr, u, W1, att1_src, att1_dst, b1,
           W2, att2_src, att2_dst, b2, Wd1, bd1, Wd2, bd2, Wd3, bd3):
    n = x.shape[0]
    e = edge_index.shape[1]
    src = edge_index[0]
    dst = edge_index[1]
    del e
    zeros16 = jnp.zeros((n, L), jnp.float32)
    zeros32 = jnp.zeros((n, HID), jnp.float32)

    u_node = jnp.broadcast_to(u, (n, u.shape[1]))
    h0 = jnp.concatenate([x, u_node], axis=-1)

    H1, a1sp, a1dp, ms1, md1 = _proj(
        h0, W1, _block_diag_att(att1_src), _block_diag_att(att1_dst))
    c1, c1vec = _shift(ms1, md1)
    exT1, dp1 = _sc_pass_a(src, dst, a1sp, a1dp, c1vec, zeros16)
    srcx = jnp.concatenate([src * HEADS + hd for hd in range(HEADS)])
    mp1 = _sc_pass_b(srcx, dst, exT1, H1.reshape(HEADS * n, HID), zeros32)
    H2, a2sp, a2dp, ms2, md2 = _combine1(
        dp1, a1sp, a1dp, c1, H1, mp1, b1[None], W2,
        _block_diag_att(att2_src), _block_diag_att(att2_dst))

    c2, c2vec = _shift(ms2, md2)
    exT2, dp2 = _sc_pass_a(src, dst, a2sp, a2dp, c2vec, zeros16)
    mp2 = _sc_pass_b(srcx, dst, exT2, H2.reshape(HEADS * n, HID), zeros32)
    hfin = _combine2(dp2, a2sp, a2dp, c2, H2, mp2, b2[None])

    hs, hd = _sc_pass_d(src, dst, hfin)
    return _decoder(hs, hd, edge_attr, Wd1, bd1, Wd2, bd2, Wd3, bd3)
